# Initial kernel scaffold; baseline (speedup 1.0000x reference)
import jax
import jax.numpy as jnp
from jax.experimental import pallas as pl


def kernel(edge_index, edge_weight, meta_edge_index, meta_edge_weight, product_idx, pert_index, batch, params):
    raise NotImplementedError("write your pallas kernel here")


# speedup vs baseline: 1.0467x; 1.0467x over previous
import jax
import jax.numpy as jnp
from jax.experimental import pallas as pl

_NUM_MET = 128
_HID = 128
_NUM_LAYERS = 2
_K_HOPS = 2
_BN_EPS = 1e-5


def _identity_pallas(x):
    def body(x_ref, o_ref):
        o_ref[...] = x_ref[...]
    return pl.pallas_call(body, out_shape=jax.ShapeDtypeStruct(x.shape, x.dtype))(x)


def _gcn_norm_j(edge_index, edge_weight, n):
    loop = jnp.arange(n, dtype=edge_index.dtype)
    ei = jnp.concatenate([edge_index, jnp.stack([loop, loop])], axis=1)
    ew = jnp.concatenate([edge_weight, jnp.ones((n,), edge_weight.dtype)])
    row, col = ei[0], ei[1]
    deg = jnp.zeros((n,), ew.dtype).at[col].add(ew)
    dis = jnp.where(deg > 0, 1.0 / jnp.sqrt(jnp.maximum(deg, 1e-12)), 0.0)
    norm = dis[row] * ew * dis[col]
    return ei, norm


def _sgconv_j(x, ei, norm, W, b):
    for _ in range(_K_HOPS):
        x = jnp.zeros_like(x).at[ei[1]].add(x[ei[0]] * norm[:, None])
    return x @ W.T + b


def _bn_eval_j(x, g, beta):
    return x / jnp.sqrt(1.0 + _BN_EPS) * g + beta


def _mlp2_j(x, p, prefix):
    x = x @ p[prefix + '_W1'].T + p[prefix + '_b1']
    x = _bn_eval_j(x, p[prefix + '_bn1_g'], p[prefix + '_bn1_b'])
    x = jax.nn.relu(x)
    x = x @ p[prefix + '_W2'].T + p[prefix + '_b2']
    x = _bn_eval_j(x, p[prefix + '_bn2_g'], p[prefix + '_bn2_b'])
    return x


def _emb_maxnorm_j(table, idx):
    v = jnp.take(table, idx, axis=0)
    nrm = jnp.linalg.norm(v, axis=-1, keepdims=True)
    return jnp.where(nrm > 1.0, v / jnp.maximum(nrm, 1e-12), v)


def kernel(edge_index, edge_weight, meta_edge_index, meta_edge_weight, product_idx, pert_index, batch, params):
    p = params
    ub = product_idx.shape[0]
    n = ub * _NUM_MET
    idx_rep = jnp.tile(jnp.arange(_NUM_MET, dtype=jnp.int32), ub)
    pos_emb = _emb_maxnorm_j(p['meta_graph_emb'], idx_rep)
    ei, norm = _gcn_norm_j(edge_index, edge_weight, n)
    for l in range(_NUM_LAYERS):
        pos_emb = _sgconv_j(pos_emb, ei, norm, p['sg_gem_%d_W' % l], p['sg_gem_%d_b' % l])
        if l < _NUM_LAYERS - 1:
            pos_emb = jax.nn.relu(pos_emb)
    base_emb = _mlp2_j(pos_emb, p, 'emb_mlp')
    base_emb = base_emb.reshape(ub, _NUM_MET, -1)
    meta_emb = _emb_maxnorm_j(p['meta_graph_emb'], jnp.arange(_NUM_MET, dtype=jnp.int32))
    mei, mnorm = _gcn_norm_j(meta_edge_index, meta_edge_weight, _NUM_MET)
    for l in range(_NUM_LAYERS):
        meta_emb = _sgconv_j(meta_emb, mei, mnorm, p['sg_meta_%d_W' % l], p['sg_meta_%d_b' % l])
        if l < _NUM_LAYERS - 1:
            meta_emb = jax.nn.relu(meta_emb)
    product_emb = jnp.take(meta_emb, product_idx, axis=0)
    product_emb = _mlp2_j(product_emb, p, 'product_mlp')
    pert_all = _emb_maxnorm_j(p['pert_emb'], jnp.arange(p['pert_emb'].shape[0], dtype=jnp.int32))
    pert_emb = jnp.take(pert_all, pert_index, axis=0).sum(axis=1)
    pert_emb = _mlp2_j(pert_emb, p, 'pert_mlp')
    x = base_emb.reshape(ub, -1)
    for i in (1, 2, 3):
        x = x @ p['flat_fc%d_W' % i].T + p['flat_fc%d_b' % i]
        x = _bn_eval_j(x, p['flat_bn%d_g' % i], p['flat_bn%d_b' % i])
        x = jax.nn.relu(x)
    base = jnp.concatenate([x, pert_emb, product_emb], axis=1)
    for i in (1, 2, 3, 4):
        base = base @ p['ff%d_W' % i].T + p['ff%d_b' % i]
        if i < 4:
            base = jax.nn.relu(base)
    out = base @ p['fc_out_W'].T + p['fc_out_b']
    return _identity_pallas(jax.nn.softmax(out, axis=1))


# speedup vs baseline: 4.4486x; 4.2501x over previous
import functools
import math

import jax
import jax.numpy as jnp
from jax import lax
from jax.experimental import pallas as pl
from jax.experimental.pallas import tpu as pltpu
from jax.experimental.pallas import tpu_sc as plsc

NC, NS, L = 2, 16, 16
NW = NC * NS
N = 8192
H = 128
NMET = 128
B = 64
E2 = 524288 + N
EPW = E2 // NW
NPB = N // NW
CH = 256
TOT = E2 + NW * 256 + NW * NW * 16
EM2P = 2560
EMW = EM2P // NW
INV = 1.0 / math.sqrt(1.0 + 1e-5)

_mesh = plsc.VectorSubcoreMesh(core_axis_name="c", subcore_axis_name="s",
                               num_cores=NC, num_subcores=NS)
_scp = pltpu.CompilerParams(needs_layout_passes=False)


def _sget(ref, idx):
    return ref[pl.ds(idx, L)][0]


@functools.partial(
    pl.kernel, mesh=_mesh, compiler_params=_scp,
    out_type=jax.ShapeDtypeStruct((NW, N), jnp.float32),
    scratch_types=[
        pltpu.VMEM((EPW,), jnp.int32),
        pltpu.VMEM((EPW,), jnp.float32),
        pltpu.VMEM((N,), jnp.float32),
    ])
def _deg_kernel(col_hbm, ew_hbm, out_hbm, col_v, ew_v, acc_v):
    c = lax.axis_index("c")
    s = lax.axis_index("s")
    wid = c * NS + s
    base = wid * EPW
    pltpu.sync_copy(col_hbm.at[pl.ds(base, EPW)], col_v)
    pltpu.sync_copy(ew_hbm.at[pl.ds(base, EPW)], ew_v)
    zero = jnp.zeros((L,), jnp.float32)

    def zb(i, carry):
        acc_v[pl.ds(i * L, L)] = zero
        return carry

    lax.fori_loop(0, N // L, zb, 0)

    def body(i, carry):
        cc = col_v[pl.ds(i * L, L)]
        w = ew_v[pl.ds(i * L, L)]
        plsc.addupdate_scatter(acc_v, [cc], w)
        return carry

    lax.fori_loop(0, EPW // L, body, 0)
    pltpu.sync_copy(acc_v, out_hbm.at[wid])


@functools.partial(
    pl.kernel, mesh=_mesh, compiler_params=_scp,
    out_type=(jax.ShapeDtypeStruct((E2,), jnp.float32),
              jax.ShapeDtypeStruct((NW, NW), jnp.float32)),
    scratch_types=[
        pltpu.VMEM((EPW,), jnp.int32),
        pltpu.VMEM((EPW,), jnp.int32),
        pltpu.VMEM((EPW,), jnp.float32),
        pltpu.VMEM((EPW,), jnp.float32),
        pltpu.VMEM((N,), jnp.float32),
        pltpu.VMEM((NW,), jnp.float32),
    ])
def _norm_kernel(row_hbm, col_hbm, ew_hbm, dis_hbm, out_hbm, cnt_hbm,
                 row_v, col_v, ew_v, nrm_v, dis_v, hist_v):
    c = lax.axis_index("c")
    s = lax.axis_index("s")
    wid = c * NS + s
    base = wid * EPW
    pltpu.sync_copy(row_hbm.at[pl.ds(base, EPW)], row_v)
    pltpu.sync_copy(col_hbm.at[pl.ds(base, EPW)], col_v)
    pltpu.sync_copy(ew_hbm.at[pl.ds(base, EPW)], ew_v)
    pltpu.sync_copy(dis_hbm, dis_v)
    zero = jnp.zeros((L,), jnp.float32)
    hist_v[pl.ds(0, L)] = zero
    hist_v[pl.ds(L, L)] = zero
    ones = jnp.ones((L,), jnp.float32)

    def body(i, carry):
        r = row_v[pl.ds(i * L, L)]
        cc = col_v[pl.ds(i * L, L)]
        w = ew_v[pl.ds(i * L, L)]
        dr = plsc.load_gather(dis_v, [r])
        dc = plsc.load_gather(dis_v, [cc])
        nrm_v[pl.ds(i * L, L)] = dr * w * dc
        bkt = lax.shift_right_logical(cc, 8)
        plsc.addupdate_scatter(hist_v, [bkt], ones)
        return carry

    lax.fori_loop(0, EPW // L, body, 0, unroll=2)
    pltpu.sync_copy(nrm_v, out_hbm.at[pl.ds(base, EPW)])
    pltpu.sync_copy(hist_v, cnt_hbm.at[wid])


@functools.partial(
    pl.kernel, mesh=_mesh, compiler_params=_scp,
    out_type=(jax.ShapeDtypeStruct((TOT,), jnp.int32),
              jax.ShapeDtypeStruct((TOT,), jnp.float32)),
    scratch_types=[
        pltpu.VMEM((EPW,), jnp.int32),
        pltpu.VMEM((EPW,), jnp.int32),
        pltpu.VMEM((EPW,), jnp.float32),
        pltpu.VMEM((EPW + L,), jnp.int32),
        pltpu.VMEM((EPW + L,), jnp.float32),
        pltpu.VMEM((NW * NW + L,), jnp.int32),
        pltpu.VMEM((NW + L,), jnp.int32),
        pltpu.VMEM((NW + L,), jnp.int32),
        pltpu.VMEM((L,), jnp.int32),
        pltpu.VMEM((L,), jnp.float32),
    ])
def _bucket_kernel(row_hbm, col_hbm, nrm_hbm, offs_hbm, tzs_hbm, tzn_hbm,
                   pk_hbm, nm_hbm,
                   row_v, col_v, nrm_v, pst_v, nst_v, offs_v, tzs_v, tzn_v,
                   zi_v, zf_v):
    c = lax.axis_index("c")
    s = lax.axis_index("s")
    wid = c * NS + s
    base = wid * EPW
    pltpu.sync_copy(row_hbm.at[pl.ds(base, EPW)], row_v)
    pltpu.sync_copy(col_hbm.at[pl.ds(base, EPW)], col_v)
    pltpu.sync_copy(nrm_hbm.at[pl.ds(base, EPW)], nrm_v)
    pltpu.sync_copy(offs_hbm, offs_v)
    pltpu.sync_copy(tzs_hbm, tzs_v)
    pltpu.sync_copy(tzn_hbm, tzn_v)
    zi_v[...] = jnp.zeros((L,), jnp.int32)
    zf_v[...] = jnp.zeros((L,), jnp.float32)

    for b in range(NW):
        def body(i, cnt):
            cc = col_v[pl.ds(i * L, L)]
            r = row_v[pl.ds(i * L, L)]
            w = nrm_v[pl.ds(i * L, L)]
            m = lax.shift_right_logical(cc, 8) == b
            pk = lax.bitwise_or(
                lax.shift_left(lax.bitwise_and(cc, NPB - 1), 13), r)
            plsc.store_compressed(pst_v.at[pl.ds(cnt, L)], pk, mask=m)
            plsc.store_compressed(nst_v.at[pl.ds(cnt, L)], w, mask=m)
            pc = plsc.all_reduce_population_count(m)
            return cnt + lax.reduce_max(pc, (0,))

        cnt = lax.fori_loop(0, EPW // L, body, 0)
        pst_v[pl.ds(cnt, L)] = jnp.zeros((L,), jnp.int32)
        nst_v[pl.ds(cnt, L)] = jnp.zeros((L,), jnp.float32)
        off = pl.multiple_of(_sget(offs_v, wid * NW + b), 16)
        n16 = (cnt + 15) // 16

        def wr(k, carry):
            pltpu.sync_copy(pst_v.at[pl.ds(k * L, L)],
                            pk_hbm.at[pl.ds(off + k * L, L)])
            pltpu.sync_copy(nst_v.at[pl.ds(k * L, L)],
                            nm_hbm.at[pl.ds(off + k * L, L)])
            return carry

        lax.fori_loop(0, n16, wr, 0)

    tz = pl.multiple_of(_sget(tzs_v, wid), 16)
    tn = _sget(tzn_v, wid)

    def tzb(k, carry):
        pltpu.sync_copy(zi_v, pk_hbm.at[pl.ds(tz + k * L, L)])
        pltpu.sync_copy(zf_v, nm_hbm.at[pl.ds(tz + k * L, L)])
        return carry

    lax.fori_loop(0, tn, tzb, 0)


@functools.partial(
    pl.kernel, mesh=_mesh, compiler_params=_scp,
    out_type=jax.ShapeDtypeStruct((N, H), jnp.float32),
    scratch_types=[
        pltpu.VMEM((NPB, H), jnp.float32),
        pltpu.VMEM((CH, H), jnp.float32),
        pltpu.VMEM((CH,), jnp.int32),
        pltpu.VMEM((2, 128), jnp.int32),
        pltpu.VMEM((CH + L,), jnp.int32),
        pltpu.VMEM((CH + L,), jnp.float32),
        pltpu.VMEM((NW + L,), jnp.int32),
        pltpu.VMEM((NW + L,), jnp.int32),
        pltpu.SemaphoreType.DMA,
    ])
def _hop_kernel(x_hbm, pk_hbm, nm_hbm, bases_hbm, nch_hbm, out_hbm,
                acc_v, rows_v, pk_v, gr_v, lx_v, nm_v, bas_v, nch_v, sem):
    c = lax.axis_index("c")
    s = lax.axis_index("s")
    wid = c * NS + s
    pltpu.sync_copy(bases_hbm, bas_v)
    pltpu.sync_copy(nch_hbm, nch_v)
    zero = jnp.zeros((L,), jnp.float32)

    def zb(i, carry):
        for q in range(H // L):
            acc_v[i, pl.ds(q * L, L)] = zero
        return carry

    lax.fori_loop(0, NPB, zb, 0)
    base = pl.multiple_of(_sget(bas_v, wid), 256)
    nch = _sget(nch_v, wid)

    def chunk(ch, carry):
        pos = pl.multiple_of(base + ch * CH, 256)
        pltpu.sync_copy(pk_hbm.at[pl.ds(pos, CH)], pk_v)
        pltpu.sync_copy(nm_hbm.at[pl.ds(pos, CH)], nm_v.at[pl.ds(0, CH)])
        for k in range(CH // L):
            pk = pk_v[pl.ds(k * L, L)]
            gr_v[k // 8, pl.ds((k % 8) * L, L)] = lax.bitwise_and(pk, N - 1)
            lx_v[pl.ds(k * L, L)] = lax.shift_right_logical(pk, 13)
        d0 = pltpu.async_copy(x_hbm.at[gr_v.at[0]], rows_v.at[pl.ds(0, 128)], sem)
        d1 = pltpu.async_copy(x_hbm.at[gr_v.at[1]], rows_v.at[pl.ds(128, 128)], sem)
        d0.wait()
        d1.wait()

        def ac(i, carry2):
            li = _sget(lx_v, i)
            w = _sget(nm_v, i)
            for q in range(H // L):
                acc_v[li, pl.ds(q * L, L)] = (
                    acc_v[li, pl.ds(q * L, L)] + rows_v[i, pl.ds(q * L, L)] * w)
            return carry2

        lax.fori_loop(0, CH, ac, 0, unroll=2)
        return carry

    lax.fori_loop(0, nch, chunk, 0)
    pltpu.sync_copy(acc_v, out_hbm.at[pl.ds(wid * NPB, NPB)])


@functools.partial(
    pl.kernel, mesh=_mesh, compiler_params=_scp,
    out_type=jax.ShapeDtypeStruct((NW, NMET * NMET), jnp.float32),
    scratch_types=[
        pltpu.VMEM((EMW,), jnp.int32),
        pltpu.VMEM((EMW,), jnp.int32),
        pltpu.VMEM((EMW,), jnp.float32),
        pltpu.VMEM((NMET * NMET,), jnp.float32),
    ])
def _meta_adj_kernel(row_hbm, col_hbm, ew_hbm, out_hbm,
                     row_v, col_v, ew_v, acc_v):
    c = lax.axis_index("c")
    s = lax.axis_index("s")
    wid = c * NS + s
    base = wid * EMW
    pltpu.sync_copy(row_hbm.at[pl.ds(base, EMW)], row_v)
    pltpu.sync_copy(col_hbm.at[pl.ds(base, EMW)], col_v)
    pltpu.sync_copy(ew_hbm.at[pl.ds(base, EMW)], ew_v)
    zero = jnp.zeros((L,), jnp.float32)

    def zb(i, carry):
        acc_v[pl.ds(i * L, L)] = zero
        return carry

    lax.fori_loop(0, NMET * NMET // L, zb, 0)

    def body(i, carry):
        r = row_v[pl.ds(i * L, L)]
        cc = col_v[pl.ds(i * L, L)]
        w = ew_v[pl.ds(i * L, L)]
        flat = cc * NMET + r
        plsc.addupdate_scatter(acc_v, [flat], w)
        return carry

    lax.fori_loop(0, EMW // L, body, 0)
    pltpu.sync_copy(acc_v, out_hbm.at[wid])


@functools.partial(
    pl.kernel, mesh=_mesh, compiler_params=_scp,
    out_type=jax.ShapeDtypeStruct((512, H), jnp.float32),
    scratch_types=[
        pltpu.VMEM((16,), jnp.int32),
        pltpu.VMEM((16, H), jnp.float32),
        pltpu.SemaphoreType.DMA,
    ])
def _pert_gather_kernel(tab_hbm, idx_hbm, out_hbm, idx_v, rows_v, sem):
    c = lax.axis_index("c")
    s = lax.axis_index("s")
    wid = c * NS + s
    pltpu.sync_copy(idx_hbm.at[pl.ds(wid * 16, 16)], idx_v)
    pltpu.async_copy(tab_hbm.at[idx_v], rows_v, sem).wait()
    pltpu.sync_copy(rows_v, out_hbm.at[pl.ds(wid * 16, 16)])


def _tc_call(body, out_shape, grid=None, in_specs=None, out_specs=None):
    kw = {}
    if grid is not None:
        kw.update(grid=grid, in_specs=in_specs, out_specs=out_specs)
    return pl.pallas_call(body, out_shape=out_shape, **kw)


def _dis_body(d_ref, o_ref):
    d = jnp.sum(d_ref[...], axis=0)
    o_ref[...] = jnp.where(d > 0, lax.rsqrt(jnp.maximum(d, 1e-12)), 0.0)


def _maxnorm_body(x_ref, o_ref):
    x = x_ref[...]
    rn = jnp.sqrt(jnp.sum(x * x, axis=1, keepdims=True))
    o_ref[...] = jnp.where(rn > 1.0, x / jnp.maximum(rn, 1e-12), x)


def _dot_t(x, w):
    return lax.dot_general(x, w, (((1,), (1,)), ((), ())),
                           preferred_element_type=jnp.float32)


def _linear_relu_body(x_ref, w_ref, b_ref, o_ref):
    z = _dot_t(x_ref[...], w_ref[...]) + b_ref[...]
    o_ref[...] = jnp.maximum(z, 0.0)


def _sg_embmlp_body(x_ref, wsg_ref, bsg_ref, w1_ref, b1_ref, g1_ref,
                    t1_ref, w2_ref, b2_ref, g2_ref, t2_ref, o_ref):
    z = _dot_t(x_ref[...], wsg_ref[...]) + bsg_ref[...]
    h = _dot_t(z, w1_ref[...]) + b1_ref[...]
    h = jnp.maximum(h * (g1_ref[...] * INV) + t1_ref[...], 0.0)
    h2 = _dot_t(h, w2_ref[...]) + b2_ref[...]
    o_ref[...] = h2 * (g2_ref[...] * INV) + t2_ref[...]


def _flat1_body(x_ref, w_ref, b_ref, g_ref, t_ref, o_ref):
    k = pl.program_id(0)

    @pl.when(k == 0)
    def _():
        o_ref[...] = jnp.zeros_like(o_ref)

    o_ref[...] += _dot_t(x_ref[...], w_ref[...])

    @pl.when(k == pl.num_programs(0) - 1)
    def _():
        z = o_ref[...] + b_ref[...]
        o_ref[...] = jnp.maximum(z * (g_ref[...] * INV) + t_ref[...], 0.0)


def _flat23_body(x_ref, w2_ref, b2_ref, g2_ref, t2_ref, w3_ref, b3_ref,
                 g3_ref, t3_ref, o_ref):
    h = _dot_t(x_ref[...], w2_ref[...]) + b2_ref[...]
    h = jnp.maximum(h * (g2_ref[...] * INV) + t2_ref[...], 0.0)
    h2 = _dot_t(h, w3_ref[...]) + b3_ref[...]
    o_ref[...] = jnp.maximum(h2 * (g3_ref[...] * INV) + t3_ref[...], 0.0)


def _meta_body(wdp_ref, mg_ref, oh_ref, w0_ref, b0_ref, w1_ref, b1_ref,
               pw1_ref, pb1_ref, pg1_ref, pt1_ref, pw2_ref, pb2_ref,
               pg2_ref, pt2_ref, o_ref):
    wd = jnp.sum(wdp_ref[...], axis=0).reshape(NMET, NMET)
    deg = jnp.sum(wd, axis=1, keepdims=True)
    dis = jnp.where(deg > 0, lax.rsqrt(jnp.maximum(deg, 1e-12)), 0.0)
    dis2 = dis * dis

    def a2(v):
        t = dis * v
        t = lax.dot_general(wd, t, (((1,), (0,)), ((), ())),
                            preferred_element_type=jnp.float32)
        t = dis2 * t
        t = lax.dot_general(wd, t, (((1,), (0,)), ((), ())),
                            preferred_element_type=jnp.float32)
        return dis * t

    m = mg_ref[...]
    m = _dot_t(a2(m), w0_ref[...]) + b0_ref[...]
    m = jnp.maximum(m, 0.0)
    m = _dot_t(a2(m), w1_ref[...]) + b1_ref[...]
    pe = lax.dot_general(oh_ref[...], m, (((1,), (0,)), ((), ())),
                         preferred_element_type=jnp.float32)
    h = _dot_t(pe, pw1_ref[...]) + pb1_ref[...]
    h = jnp.maximum(h * (pg1_ref[...] * INV) + pt1_ref[...], 0.0)
    h2 = _dot_t(h, pw2_ref[...]) + pb2_ref[...]
    o_ref[...] = h2 * (pg2_ref[...] * INV) + pt2_ref[...]


def _pert_body(g_ref, pw1_ref, pb1_ref, pg1_ref, pt1_ref, pw2_ref, pb2_ref,
               pg2_ref, pt2_ref, o_ref):
    v = g_ref[...]
    rn = jnp.sqrt(jnp.sum(v * v, axis=1, keepdims=True))
    v = jnp.where(rn > 1.0, v / jnp.maximum(rn, 1e-12), v)
    r = lax.broadcasted_iota(jnp.int32, (B, 512), 0)
    k = lax.broadcasted_iota(jnp.int32, (B, 512), 1)
    sel = jnp.where(lax.div(k, 8) == r, 1.0, 0.0)
    sv = lax.dot_general(sel, v, (((1,), (0,)), ((), ())),
                         preferred_element_type=jnp.float32)
    h = _dot_t(sv, pw1_ref[...]) + pb1_ref[...]
    h = jnp.maximum(h * (pg1_ref[...] * INV) + pt1_ref[...], 0.0)
    h2 = _dot_t(h, pw2_ref[...]) + pb2_ref[...]
    o_ref[...] = h2 * (pg2_ref[...] * INV) + pt2_ref[...]


def _final_body(x_ref, pe_ref, pr_ref, w1_ref, b1_ref, w2_ref, b2_ref,
                w3_ref, b3_ref, w4_ref, b4_ref, wo_ref, bo_ref, o_ref):
    x = jnp.concatenate([x_ref[...], pe_ref[...], pr_ref[...]], axis=1)
    h = jnp.maximum(_dot_t(x, w1_ref[...]) + b1_ref[...], 0.0)
    h = jnp.maximum(_dot_t(h, w2_ref[...]) + b2_ref[...], 0.0)
    h = jnp.maximum(_dot_t(h, w3_ref[...]) + b3_ref[...], 0.0)
    h = _dot_t(h, w4_ref[...]) + b4_ref[...]
    z = _dot_t(h, wo_ref[...]) + bo_ref[...]
    m = jnp.max(z, axis=1, keepdims=True)
    e = jnp.exp(z - m)
    o_ref[...] = e / jnp.sum(e, axis=1, keepdims=True)


def kernel(edge_index, edge_weight, meta_edge_index, meta_edge_weight,
           product_idx, pert_index, batch, params):
    p = params
    f32 = jnp.float32

    ar = jnp.arange(N, dtype=jnp.int32)
    row = jnp.concatenate([edge_index[0], ar])
    col = jnp.concatenate([edge_index[1], ar])
    ew2 = jnp.concatenate([edge_weight, jnp.ones((N,), f32)])

    degp = _deg_kernel(col, ew2)
    dis64 = _tc_call(_dis_body, jax.ShapeDtypeStruct((N // 128, 128), f32))(
        degp.reshape(NW, N // 128, 128))
    dis = dis64.reshape(N)
    norm, cntf = _norm_kernel(row, col, ew2, dis)

    cnts = cntf.astype(jnp.int32)
    c16 = ((cnts + 15) // 16) * 16
    tot16 = jnp.sum(c16, axis=0)
    caps = ((tot16 + 255) // 256) * 256
    bases = jnp.concatenate([jnp.zeros((1,), jnp.int32),
                             jnp.cumsum(caps)[:-1].astype(jnp.int32)])
    excl = jnp.concatenate([jnp.zeros((1, NW), jnp.int32),
                            jnp.cumsum(c16, axis=0)[:-1].astype(jnp.int32)],
                           axis=0)
    padL = jnp.zeros((L,), jnp.int32)
    offs = jnp.concatenate([(bases[None, :] + excl).reshape(-1), padL])
    tzs = jnp.concatenate([bases + tot16, padL])
    tzn = jnp.concatenate([(caps - tot16) // 16, padL])
    nch = jnp.concatenate([caps // CH, padL])
    bases_p = jnp.concatenate([bases, padL])

    pk, nm = _bucket_kernel(row, col, norm, offs, tzs, tzn)

    mg_mx = _tc_call(_maxnorm_body, jax.ShapeDtypeStruct((NMET, H), f32))(
        p['meta_graph_emb'])
    x0 = jnp.broadcast_to(mg_mx[None], (B, NMET, H)).reshape(N, H)

    def r1(v):
        return v.reshape(1, -1)

    h1 = _hop_kernel(x0, pk, nm, bases_p, nch)
    h2 = _hop_kernel(h1, pk, nm, bases_p, nch)
    grid16 = (16,)
    bs_x = pl.BlockSpec((512, H), lambda i: (i, 0))
    bs_w = pl.BlockSpec((H, H), lambda i: (0, 0))
    bs_b = pl.BlockSpec((1, H), lambda i: (0, 0))
    x1 = _tc_call(_linear_relu_body, jax.ShapeDtypeStruct((N, H), f32),
                  grid=grid16, in_specs=[bs_x, bs_w, bs_b],
                  out_specs=bs_x)(h2, p['sg_gem_0_W'], r1(p['sg_gem_0_b']))
    h3 = _hop_kernel(x1, pk, nm, bases_p, nch)
    h4 = _hop_kernel(h3, pk, nm, bases_p, nch)
    base_emb = _tc_call(
        _sg_embmlp_body, jax.ShapeDtypeStruct((N, H), f32), grid=grid16,
        in_specs=[bs_x, bs_w, bs_b] + [bs_w, bs_b, bs_b, bs_b] * 2,
        out_specs=bs_x)(
        h4, p['sg_gem_1_W'], r1(p['sg_gem_1_b']),
        p['emb_mlp_W1'], r1(p['emb_mlp_b1']), r1(p['emb_mlp_bn1_g']), r1(p['emb_mlp_bn1_b']),
        p['emb_mlp_W2'], r1(p['emb_mlp_b2']), r1(p['emb_mlp_bn2_g']), r1(p['emb_mlp_bn2_b']))

    xflat = base_emb.reshape(B, NMET * H)
    fl1 = _tc_call(
        _flat1_body, jax.ShapeDtypeStruct((B, 1024), f32), grid=(32,),
        in_specs=[pl.BlockSpec((B, 512), lambda k: (0, k)),
                  pl.BlockSpec((1024, 512), lambda k: (0, k)),
                  pl.BlockSpec((1, 1024), lambda k: (0, 0)),
                  pl.BlockSpec((1, 1024), lambda k: (0, 0)),
                  pl.BlockSpec((1, 1024), lambda k: (0, 0))],
        out_specs=pl.BlockSpec((B, 1024), lambda k: (0, 0)))(
        xflat, p['flat_fc1_W'], r1(p['flat_fc1_b']),
        r1(p['flat_bn1_g']), r1(p['flat_bn1_b']))
    fl3 = _tc_call(_flat23_body, jax.ShapeDtypeStruct((B, H), f32))(
        fl1, p['flat_fc2_W'], r1(p['flat_fc2_b']), r1(p['flat_bn2_g']), r1(p['flat_bn2_b']),
        p['flat_fc3_W'], r1(p['flat_fc3_b']), r1(p['flat_bn3_g']), r1(p['flat_bn3_b']))

    mar = jnp.arange(NMET, dtype=jnp.int32)
    pad = EM2P - (meta_edge_index.shape[1] + NMET)
    mrow = jnp.concatenate([meta_edge_index[0], mar, jnp.zeros((pad,), jnp.int32)])
    mcol = jnp.concatenate([meta_edge_index[1], mar, jnp.zeros((pad,), jnp.int32)])
    mew = jnp.concatenate([meta_edge_weight, jnp.ones((NMET,), f32),
                           jnp.zeros((pad,), f32)])
    wdp = _meta_adj_kernel(mrow, mcol, mew)
    onehot = (product_idx[:, None] == mar[None, :]).astype(f32)
    prod = _tc_call(_meta_body, jax.ShapeDtypeStruct((B, H), f32))(
        wdp.reshape(NW, NMET, NMET), mg_mx, onehot,
        p['sg_meta_0_W'], r1(p['sg_meta_0_b']),
        p['sg_meta_1_W'], r1(p['sg_meta_1_b']),
        p['product_mlp_W1'], r1(p['product_mlp_b1']),
        r1(p['product_mlp_bn1_g']), r1(p['product_mlp_bn1_b']),
        p['product_mlp_W2'], r1(p['product_mlp_b2']),
        r1(p['product_mlp_bn2_g']), r1(p['product_mlp_bn2_b']))

    pg = _pert_gather_kernel(p['pert_emb'], pert_index.reshape(512))
    pert = _tc_call(_pert_body, jax.ShapeDtypeStruct((B, H), f32))(
        pg,
        p['pert_mlp_W1'], r1(p['pert_mlp_b1']),
        r1(p['pert_mlp_bn1_g']), r1(p['pert_mlp_bn1_b']),
        p['pert_mlp_W2'], r1(p['pert_mlp_b2']),
        r1(p['pert_mlp_bn2_g']), r1(p['pert_mlp_bn2_b']))

    wo = jnp.zeros((128, 256), f32).at[:2].set(p['fc_out_W'])
    bo = jnp.full((1, 128), -1e30, f32).at[0, :2].set(p['fc_out_b'])
    out = _tc_call(_final_body, jax.ShapeDtypeStruct((B, 128), f32))(
        fl3, pert, prod,
        p['ff1_W'], r1(p['ff1_b']), p['ff2_W'], r1(p['ff2_b']),
        p['ff3_W'], r1(p['ff3_b']), p['ff4_W'], r1(p['ff4_b']),
        wo, bo)
    return out[:, :2]


# speedup vs baseline: 5.6653x; 1.2735x over previous
import functools
import math

import jax
import jax.numpy as jnp
from jax import lax
from jax.experimental import pallas as pl
from jax.experimental.pallas import tpu as pltpu
from jax.experimental.pallas import tpu_sc as plsc

NC, NS, L = 2, 16, 16
NW = NC * NS
N = 8192
H = 128
NMET = 128
B = 64
E2 = 524288 + N
EPW = E2 // NW
NPB = N // NW
CH = 256
TOT = E2 + NW * 256 + NW * NW * 16
EM2P = 2560
EMW = EM2P // NW
INV = 1.0 / math.sqrt(1.0 + 1e-5)

_mesh = plsc.VectorSubcoreMesh(core_axis_name="c", subcore_axis_name="s",
                               num_cores=NC, num_subcores=NS)
_scp = pltpu.CompilerParams(needs_layout_passes=False)


def _sget(ref, idx):
    return ref[pl.ds(idx, L)][0]


@functools.partial(
    pl.kernel, mesh=_mesh, compiler_params=_scp,
    out_type=jax.ShapeDtypeStruct((NW, N), jnp.float32),
    scratch_types=[
        pltpu.VMEM((EPW,), jnp.int32),
        pltpu.VMEM((EPW,), jnp.float32),
        pltpu.VMEM((N,), jnp.float32),
    ])
def _deg_kernel(col_hbm, ew_hbm, out_hbm, col_v, ew_v, acc_v):
    c = lax.axis_index("c")
    s = lax.axis_index("s")
    wid = c * NS + s
    base = wid * EPW
    pltpu.sync_copy(col_hbm.at[pl.ds(base, EPW)], col_v)
    pltpu.sync_copy(ew_hbm.at[pl.ds(base, EPW)], ew_v)
    zero = jnp.zeros((L,), jnp.float32)

    def zb(i, carry):
        acc_v[pl.ds(i * L, L)] = zero
        return carry

    lax.fori_loop(0, N // L, zb, 0)

    def body(i, carry):
        cc = col_v[pl.ds(i * L, L)]
        w = ew_v[pl.ds(i * L, L)]
        plsc.addupdate_scatter(acc_v, [cc], w)
        return carry

    lax.fori_loop(0, EPW // L, body, 0)
    pltpu.sync_copy(acc_v, out_hbm.at[wid])


@functools.partial(
    pl.kernel, mesh=_mesh, compiler_params=_scp,
    out_type=(jax.ShapeDtypeStruct((E2,), jnp.float32),
              jax.ShapeDtypeStruct((NW, NW), jnp.float32)),
    scratch_types=[
        pltpu.VMEM((EPW,), jnp.int32),
        pltpu.VMEM((EPW,), jnp.int32),
        pltpu.VMEM((EPW,), jnp.float32),
        pltpu.VMEM((EPW,), jnp.float32),
        pltpu.VMEM((N,), jnp.float32),
        pltpu.VMEM((NW,), jnp.float32),
    ])
def _norm_kernel(row_hbm, col_hbm, ew_hbm, dis_hbm, out_hbm, cnt_hbm,
                 row_v, col_v, ew_v, nrm_v, dis_v, hist_v):
    c = lax.axis_index("c")
    s = lax.axis_index("s")
    wid = c * NS + s
    base = wid * EPW
    pltpu.sync_copy(row_hbm.at[pl.ds(base, EPW)], row_v)
    pltpu.sync_copy(col_hbm.at[pl.ds(base, EPW)], col_v)
    pltpu.sync_copy(ew_hbm.at[pl.ds(base, EPW)], ew_v)
    pltpu.sync_copy(dis_hbm, dis_v)
    zero = jnp.zeros((L,), jnp.float32)
    hist_v[pl.ds(0, L)] = zero
    hist_v[pl.ds(L, L)] = zero
    ones = jnp.ones((L,), jnp.float32)

    def body(i, carry):
        r = row_v[pl.ds(i * L, L)]
        cc = col_v[pl.ds(i * L, L)]
        w = ew_v[pl.ds(i * L, L)]
        dr = plsc.load_gather(dis_v, [r])
        dc = plsc.load_gather(dis_v, [cc])
        nrm_v[pl.ds(i * L, L)] = dr * w * dc
        bkt = lax.shift_right_logical(cc, 8)
        plsc.addupdate_scatter(hist_v, [bkt], ones)
        return carry

    lax.fori_loop(0, EPW // L, body, 0, unroll=2)
    pltpu.sync_copy(nrm_v, out_hbm.at[pl.ds(base, EPW)])
    pltpu.sync_copy(hist_v, cnt_hbm.at[wid])


@functools.partial(
    pl.kernel, mesh=_mesh, compiler_params=_scp,
    out_type=(jax.ShapeDtypeStruct((TOT,), jnp.int32),
              jax.ShapeDtypeStruct((TOT,), jnp.float32)),
    scratch_types=[
        pltpu.VMEM((EPW,), jnp.int32),
        pltpu.VMEM((EPW,), jnp.int32),
        pltpu.VMEM((EPW,), jnp.float32),
        pltpu.VMEM((EPW + L,), jnp.int32),
        pltpu.VMEM((EPW + L,), jnp.float32),
        pltpu.VMEM((NW * NW + L,), jnp.int32),
        pltpu.VMEM((NW + L,), jnp.int32),
        pltpu.VMEM((NW + L,), jnp.int32),
        pltpu.VMEM((L,), jnp.int32),
        pltpu.VMEM((L,), jnp.float32),
    ])
def _bucket_kernel(row_hbm, col_hbm, nrm_hbm, offs_hbm, tzs_hbm, tzn_hbm,
                   pk_hbm, nm_hbm,
                   row_v, col_v, nrm_v, pst_v, nst_v, offs_v, tzs_v, tzn_v,
                   zi_v, zf_v):
    c = lax.axis_index("c")
    s = lax.axis_index("s")
    wid = c * NS + s
    base = wid * EPW
    pltpu.sync_copy(row_hbm.at[pl.ds(base, EPW)], row_v)
    pltpu.sync_copy(col_hbm.at[pl.ds(base, EPW)], col_v)
    pltpu.sync_copy(nrm_hbm.at[pl.ds(base, EPW)], nrm_v)
    pltpu.sync_copy(offs_hbm, offs_v)
    pltpu.sync_copy(tzs_hbm, tzs_v)
    pltpu.sync_copy(tzn_hbm, tzn_v)
    zi_v[...] = jnp.zeros((L,), jnp.int32)
    zf_v[...] = jnp.zeros((L,), jnp.float32)

    for b in range(NW):
        def body(i, cnt):
            cc = col_v[pl.ds(i * L, L)]
            r = row_v[pl.ds(i * L, L)]
            w = nrm_v[pl.ds(i * L, L)]
            m = lax.shift_right_logical(cc, 8) == b
            pk = lax.bitwise_or(
                lax.shift_left(lax.bitwise_and(cc, NPB - 1), 13), r)
            plsc.store_compressed(pst_v.at[pl.ds(cnt, L)], pk, mask=m)
            plsc.store_compressed(nst_v.at[pl.ds(cnt, L)], w, mask=m)
            pc = plsc.all_reduce_population_count(m)
            return cnt + lax.reduce_max(pc, (0,))

        cnt = lax.fori_loop(0, EPW // L, body, 0)
        pst_v[pl.ds(cnt, L)] = jnp.zeros((L,), jnp.int32)
        nst_v[pl.ds(cnt, L)] = jnp.zeros((L,), jnp.float32)
        off = pl.multiple_of(_sget(offs_v, wid * NW + b), 16)
        n16 = (cnt + 15) // 16

        def wr(k, carry):
            pltpu.sync_copy(pst_v.at[pl.ds(k * L, L)],
                            pk_hbm.at[pl.ds(off + k * L, L)])
            pltpu.sync_copy(nst_v.at[pl.ds(k * L, L)],
                            nm_hbm.at[pl.ds(off + k * L, L)])
            return carry

        lax.fori_loop(0, n16, wr, 0)

    tz = pl.multiple_of(_sget(tzs_v, wid), 16)
    tn = _sget(tzn_v, wid)

    def tzb(k, carry):
        pltpu.sync_copy(zi_v, pk_hbm.at[pl.ds(tz + k * L, L)])
        pltpu.sync_copy(zf_v, nm_hbm.at[pl.ds(tz + k * L, L)])
        return carry

    lax.fori_loop(0, tn, tzb, 0)


@functools.partial(
    pl.kernel, mesh=_mesh, compiler_params=_scp,
    out_type=jax.ShapeDtypeStruct((N, H), jnp.float32),
    scratch_types=[
        pltpu.VMEM((NPB, H), jnp.float32),
        pltpu.VMEM((CH, H), jnp.float32),
        pltpu.VMEM((CH, H), jnp.float32),
        pltpu.VMEM((CH,), jnp.int32),
        pltpu.VMEM((CH,), jnp.int32),
        pltpu.VMEM((2, 128), jnp.int32),
        pltpu.VMEM((2, 128), jnp.int32),
        pltpu.VMEM((CH,), jnp.int32),
        pltpu.VMEM((CH,), jnp.int32),
        pltpu.VMEM((CH,), jnp.float32),
        pltpu.VMEM((CH,), jnp.float32),
        pltpu.VMEM((NW + L,), jnp.int32),
        pltpu.VMEM((NW + L,), jnp.int32),
        pltpu.SemaphoreType.DMA,
        pltpu.SemaphoreType.DMA,
    ])
def _hop_kernel(x_hbm, pk_hbm, nm_hbm, bases_hbm, nch_hbm, out_hbm,
                acc_v, rows0_v, rows1_v, pk0_v, pk1_v, gr0_v, gr1_v,
                lx0_v, lx1_v, nm0_v, nm1_v, bas_v, nch_v, sem0, sem1):
    c = lax.axis_index("c")
    s = lax.axis_index("s")
    wid = c * NS + s
    pltpu.sync_copy(bases_hbm, bas_v)
    pltpu.sync_copy(nch_hbm, nch_v)
    zero = jnp.zeros((L,), jnp.float32)

    def zb(i, carry):
        for q in range(H // L):
            acc_v[i, pl.ds(q * L, L)] = zero
        return carry

    lax.fori_loop(0, NPB, zb, 0)
    base = pl.multiple_of(_sget(bas_v, wid), 256)
    nch = _sget(nch_v, wid)

    bufs = ((rows0_v, pk0_v, gr0_v, lx0_v, nm0_v, sem0),
            (rows1_v, pk1_v, gr1_v, lx1_v, nm1_v, sem1))

    def stage(par, ch):
        rows_v, pk_v, gr_v, lx_v, nm_v, sem = bufs[par]
        pos = pl.multiple_of(base + ch * CH, 256)
        pltpu.sync_copy(pk_hbm.at[pl.ds(pos, CH)], pk_v)
        pltpu.sync_copy(nm_hbm.at[pl.ds(pos, CH)], nm_v)
        for k in range(CH // L):
            pk = pk_v[pl.ds(k * L, L)]
            gr_v[k // 8, pl.ds((k % 8) * L, L)] = lax.bitwise_and(pk, N - 1)
            lx_v[pl.ds(k * L, L)] = lax.shift_right_logical(pk, 13)
        d0 = pltpu.async_copy(x_hbm.at[gr_v.at[0]], rows_v.at[pl.ds(0, 128)], sem)
        d1 = pltpu.async_copy(x_hbm.at[gr_v.at[1]], rows_v.at[pl.ds(128, 128)], sem)
        return d0, d1

    def drain_accum(par):
        rows_v, pk_v, gr_v, lx_v, nm_v, sem = bufs[par]
        pltpu.make_async_copy(x_hbm.at[gr_v.at[0]], rows_v.at[pl.ds(0, 128)], sem).wait()
        pltpu.make_async_copy(x_hbm.at[gr_v.at[1]], rows_v.at[pl.ds(128, 128)], sem).wait()

        def ac(g, carry2):
            lv = lx_v[pl.ds(g * L, L)]
            wv = nm_v[pl.ds(g * L, L)]
            for j in range(L):
                i = g * L + j
                li = lv[j]
                w = wv[j]
                for q in range(H // L):
                    acc_v[li, pl.ds(q * L, L)] = (
                        acc_v[li, pl.ds(q * L, L)]
                        + rows_v[i, pl.ds(q * L, L)] * w)
            return carry2

        lax.fori_loop(0, CH // L, ac, 0)

    stage(0, 0)

    def pair(t, carry):
        stage(1, 2 * t + 1)
        drain_accum(0)

        @pl.when(2 * t + 2 < nch)
        def _():
            stage(0, 2 * t + 2)
        drain_accum(1)
        return carry

    npair = nch // 2

    def tail(_t, carry):
        return carry

    lax.fori_loop(0, npair, pair, 0)

    @pl.when(lax.rem(nch, 2) == 1)
    def _():
        @pl.when(nch > 1)
        def _():
            pass
        drain_accum(0)

    pltpu.sync_copy(acc_v, out_hbm.at[pl.ds(wid * NPB, NPB)])


@functools.partial(
    pl.kernel, mesh=_mesh, compiler_params=_scp,
    out_type=jax.ShapeDtypeStruct((NW, NMET * NMET), jnp.float32),
    scratch_types=[
        pltpu.VMEM((EMW,), jnp.int32),
        pltpu.VMEM((EMW,), jnp.int32),
        pltpu.VMEM((EMW,), jnp.float32),
        pltpu.VMEM((NMET * NMET,), jnp.float32),
    ])
def _meta_adj_kernel(row_hbm, col_hbm, ew_hbm, out_hbm,
                     row_v, col_v, ew_v, acc_v):
    c = lax.axis_index("c")
    s = lax.axis_index("s")
    wid = c * NS + s
    base = wid * EMW
    pltpu.sync_copy(row_hbm.at[pl.ds(base, EMW)], row_v)
    pltpu.sync_copy(col_hbm.at[pl.ds(base, EMW)], col_v)
    pltpu.sync_copy(ew_hbm.at[pl.ds(base, EMW)], ew_v)
    zero = jnp.zeros((L,), jnp.float32)

    def zb(i, carry):
        acc_v[pl.ds(i * L, L)] = zero
        return carry

    lax.fori_loop(0, NMET * NMET // L, zb, 0)

    def body(i, carry):
        r = row_v[pl.ds(i * L, L)]
        cc = col_v[pl.ds(i * L, L)]
        w = ew_v[pl.ds(i * L, L)]
        flat = cc * NMET + r
        plsc.addupdate_scatter(acc_v, [flat], w)
        return carry

    lax.fori_loop(0, EMW // L, body, 0)
    pltpu.sync_copy(acc_v, out_hbm.at[wid])


@functools.partial(
    pl.kernel, mesh=_mesh, compiler_params=_scp,
    out_type=jax.ShapeDtypeStruct((512, H), jnp.float32),
    scratch_types=[
        pltpu.VMEM((16,), jnp.int32),
        pltpu.VMEM((16, H), jnp.float32),
        pltpu.SemaphoreType.DMA,
    ])
def _pert_gather_kernel(tab_hbm, idx_hbm, out_hbm, idx_v, rows_v, sem):
    c = lax.axis_index("c")
    s = lax.axis_index("s")
    wid = c * NS + s
    pltpu.sync_copy(idx_hbm.at[pl.ds(wid * 16, 16)], idx_v)
    pltpu.async_copy(tab_hbm.at[idx_v], rows_v, sem).wait()
    pltpu.sync_copy(rows_v, out_hbm.at[pl.ds(wid * 16, 16)])


def _tc_call(body, out_shape, grid=None, in_specs=None, out_specs=None):
    kw = {}
    if grid is not None:
        kw.update(grid=grid, in_specs=in_specs, out_specs=out_specs)
    return pl.pallas_call(body, out_shape=out_shape, **kw)


def _dis_body(d_ref, o_ref):
    d = jnp.sum(d_ref[...], axis=0)
    o_ref[...] = jnp.where(d > 0, lax.rsqrt(jnp.maximum(d, 1e-12)), 0.0)


def _maxnorm_body(x_ref, o_ref):
    x = x_ref[...]
    rn = jnp.sqrt(jnp.sum(x * x, axis=1, keepdims=True))
    o_ref[...] = jnp.where(rn > 1.0, x / jnp.maximum(rn, 1e-12), x)


def _dot_t(x, w):
    return lax.dot_general(x, w, (((1,), (1,)), ((), ())),
                           preferred_element_type=jnp.float32)


def _linear_relu_body(x_ref, w_ref, b_ref, o_ref):
    z = _dot_t(x_ref[...], w_ref[...]) + b_ref[...]
    o_ref[...] = jnp.maximum(z, 0.0)


def _sg_embmlp_body(x_ref, wsg_ref, bsg_ref, w1_ref, b1_ref, g1_ref,
                    t1_ref, w2_ref, b2_ref, g2_ref, t2_ref, o_ref):
    z = _dot_t(x_ref[...], wsg_ref[...]) + bsg_ref[...]
    h = _dot_t(z, w1_ref[...]) + b1_ref[...]
    h = jnp.maximum(h * (g1_ref[...] * INV) + t1_ref[...], 0.0)
    h2 = _dot_t(h, w2_ref[...]) + b2_ref[...]
    o_ref[...] = h2 * (g2_ref[...] * INV) + t2_ref[...]


def _flat1_body(x_ref, w_ref, b_ref, g_ref, t_ref, o_ref):
    k = pl.program_id(0)

    @pl.when(k == 0)
    def _():
        o_ref[...] = jnp.zeros_like(o_ref)

    o_ref[...] += _dot_t(x_ref[...], w_ref[...])

    @pl.when(k == pl.num_programs(0) - 1)
    def _():
        z = o_ref[...] + b_ref[...]
        o_ref[...] = jnp.maximum(z * (g_ref[...] * INV) + t_ref[...], 0.0)


def _flat23_body(x_ref, w2_ref, b2_ref, g2_ref, t2_ref, w3_ref, b3_ref,
                 g3_ref, t3_ref, o_ref):
    h = _dot_t(x_ref[...], w2_ref[...]) + b2_ref[...]
    h = jnp.maximum(h * (g2_ref[...] * INV) + t2_ref[...], 0.0)
    h2 = _dot_t(h, w3_ref[...]) + b3_ref[...]
    o_ref[...] = jnp.maximum(h2 * (g3_ref[...] * INV) + t3_ref[...], 0.0)


def _meta_body(wdp_ref, mg_ref, oh_ref, w0_ref, b0_ref, w1_ref, b1_ref,
               pw1_ref, pb1_ref, pg1_ref, pt1_ref, pw2_ref, pb2_ref,
               pg2_ref, pt2_ref, o_ref):
    wd = jnp.sum(wdp_ref[...], axis=0).reshape(NMET, NMET)
    deg = jnp.sum(wd, axis=1, keepdims=True)
    dis = jnp.where(deg > 0, lax.rsqrt(jnp.maximum(deg, 1e-12)), 0.0)
    dis2 = dis * dis

    def a2(v):
        t = dis * v
        t = lax.dot_general(wd, t, (((1,), (0,)), ((), ())),
                            preferred_element_type=jnp.float32)
        t = dis2 * t
        t = lax.dot_general(wd, t, (((1,), (0,)), ((), ())),
                            preferred_element_type=jnp.float32)
        return dis * t

    m = mg_ref[...]
    m = _dot_t(a2(m), w0_ref[...]) + b0_ref[...]
    m = jnp.maximum(m, 0.0)
    m = _dot_t(a2(m), w1_ref[...]) + b1_ref[...]
    pe = lax.dot_general(oh_ref[...], m, (((1,), (0,)), ((), ())),
                         preferred_element_type=jnp.float32)
    h = _dot_t(pe, pw1_ref[...]) + pb1_ref[...]
    h = jnp.maximum(h * (pg1_ref[...] * INV) + pt1_ref[...], 0.0)
    h2 = _dot_t(h, pw2_ref[...]) + pb2_ref[...]
    o_ref[...] = h2 * (pg2_ref[...] * INV) + pt2_ref[...]


def _pert_body(g_ref, pw1_ref, pb1_ref, pg1_ref, pt1_ref, pw2_ref, pb2_ref,
               pg2_ref, pt2_ref, o_ref):
    v = g_ref[...]
    rn = jnp.sqrt(jnp.sum(v * v, axis=1, keepdims=True))
    v = jnp.where(rn > 1.0, v / jnp.maximum(rn, 1e-12), v)
    r = lax.broadcasted_iota(jnp.int32, (B, 512), 0)
    k = lax.broadcasted_iota(jnp.int32, (B, 512), 1)
    sel = jnp.where(lax.div(k, 8) == r, 1.0, 0.0)
    sv = lax.dot_general(sel, v, (((1,), (0,)), ((), ())),
                         preferred_element_type=jnp.float32)
    h = _dot_t(sv, pw1_ref[...]) + pb1_ref[...]
    h = jnp.maximum(h * (pg1_ref[...] * INV) + pt1_ref[...], 0.0)
    h2 = _dot_t(h, pw2_ref[...]) + pb2_ref[...]
    o_ref[...] = h2 * (pg2_ref[...] * INV) + pt2_ref[...]


def _final_body(x_ref, pe_ref, pr_ref, w1_ref, b1_ref, w2_ref, b2_ref,
                w3_ref, b3_ref, w4_ref, b4_ref, wo_ref, bo_ref, o_ref):
    x = jnp.concatenate([x_ref[...], pe_ref[...], pr_ref[...]], axis=1)
    h = jnp.maximum(_dot_t(x, w1_ref[...]) + b1_ref[...], 0.0)
    h = jnp.maximum(_dot_t(h, w2_ref[...]) + b2_ref[...], 0.0)
    h = jnp.maximum(_dot_t(h, w3_ref[...]) + b3_ref[...], 0.0)
    h = _dot_t(h, w4_ref[...]) + b4_ref[...]
    z = _dot_t(h, wo_ref[...]) + bo_ref[...]
    m = jnp.max(z, axis=1, keepdims=True)
    e = jnp.exp(z - m)
    o_ref[...] = e / jnp.sum(e, axis=1, keepdims=True)


def kernel(edge_index, edge_weight, meta_edge_index, meta_edge_weight,
           product_idx, pert_index, batch, params):
    p = params
    f32 = jnp.float32

    ar = jnp.arange(N, dtype=jnp.int32)
    row = jnp.concatenate([edge_index[0], ar])
    col = jnp.concatenate([edge_index[1], ar])
    ew2 = jnp.concatenate([edge_weight, jnp.ones((N,), f32)])

    degp = _deg_kernel(col, ew2)
    dis64 = _tc_call(_dis_body, jax.ShapeDtypeStruct((N // 128, 128), f32))(
        degp.reshape(NW, N // 128, 128))
    dis = dis64.reshape(N)
    norm, cntf = _norm_kernel(row, col, ew2, dis)

    cnts = cntf.astype(jnp.int32)
    c16 = ((cnts + 15) // 16) * 16
    tot16 = jnp.sum(c16, axis=0)
    caps = ((tot16 + 255) // 256) * 256
    bases = jnp.concatenate([jnp.zeros((1,), jnp.int32),
                             jnp.cumsum(caps)[:-1].astype(jnp.int32)])
    excl = jnp.concatenate([jnp.zeros((1, NW), jnp.int32),
                            jnp.cumsum(c16, axis=0)[:-1].astype(jnp.int32)],
                           axis=0)
    padL = jnp.zeros((L,), jnp.int32)
    offs = jnp.concatenate([(bases[None, :] + excl).reshape(-1), padL])
    tzs = jnp.concatenate([bases + tot16, padL])
    tzn = jnp.concatenate([(caps - tot16) // 16, padL])
    nch = jnp.concatenate([caps // CH, padL])
    bases_p = jnp.concatenate([bases, padL])

    pk, nm = _bucket_kernel(row, col, norm, offs, tzs, tzn)

    mg_mx = _tc_call(_maxnorm_body, jax.ShapeDtypeStruct((NMET, H), f32))(
        p['meta_graph_emb'])
    x0 = jnp.broadcast_to(mg_mx[None], (B, NMET, H)).reshape(N, H)

    def r1(v):
        return v.reshape(1, -1)

    h1 = _hop_kernel(x0, pk, nm, bases_p, nch)
    h2 = _hop_kernel(h1, pk, nm, bases_p, nch)
    grid16 = (16,)
    bs_x = pl.BlockSpec((512, H), lambda i: (i, 0))
    bs_w = pl.BlockSpec((H, H), lambda i: (0, 0))
    bs_b = pl.BlockSpec((1, H), lambda i: (0, 0))
    x1 = _tc_call(_linear_relu_body, jax.ShapeDtypeStruct((N, H), f32),
                  grid=grid16, in_specs=[bs_x, bs_w, bs_b],
                  out_specs=bs_x)(h2, p['sg_gem_0_W'], r1(p['sg_gem_0_b']))
    h3 = _hop_kernel(x1, pk, nm, bases_p, nch)
    h4 = _hop_kernel(h3, pk, nm, bases_p, nch)
    base_emb = _tc_call(
        _sg_embmlp_body, jax.ShapeDtypeStruct((N, H), f32), grid=grid16,
        in_specs=[bs_x, bs_w, bs_b] + [bs_w, bs_b, bs_b, bs_b] * 2,
        out_specs=bs_x)(
        h4, p['sg_gem_1_W'], r1(p['sg_gem_1_b']),
        p['emb_mlp_W1'], r1(p['emb_mlp_b1']), r1(p['emb_mlp_bn1_g']), r1(p['emb_mlp_bn1_b']),
        p['emb_mlp_W2'], r1(p['emb_mlp_b2']), r1(p['emb_mlp_bn2_g']), r1(p['emb_mlp_bn2_b']))

    xflat = base_emb.reshape(B, NMET * H)
    fl1 = _tc_call(
        _flat1_body, jax.ShapeDtypeStruct((B, 1024), f32), grid=(32,),
        in_specs=[pl.BlockSpec((B, 512), lambda k: (0, k)),
                  pl.BlockSpec((1024, 512), lambda k: (0, k)),
                  pl.BlockSpec((1, 1024), lambda k: (0, 0)),
                  pl.BlockSpec((1, 1024), lambda k: (0, 0)),
                  pl.BlockSpec((1, 1024), lambda k: (0, 0))],
        out_specs=pl.BlockSpec((B, 1024), lambda k: (0, 0)))(
        xflat, p['flat_fc1_W'], r1(p['flat_fc1_b']),
        r1(p['flat_bn1_g']), r1(p['flat_bn1_b']))
    fl3 = _tc_call(_flat23_body, jax.ShapeDtypeStruct((B, H), f32))(
        fl1, p['flat_fc2_W'], r1(p['flat_fc2_b']), r1(p['flat_bn2_g']), r1(p['flat_bn2_b']),
        p['flat_fc3_W'], r1(p['flat_fc3_b']), r1(p['flat_bn3_g']), r1(p['flat_bn3_b']))

    mar = jnp.arange(NMET, dtype=jnp.int32)
    pad = EM2P - (meta_edge_index.shape[1] + NMET)
    mrow = jnp.concatenate([meta_edge_index[0], mar, jnp.zeros((pad,), jnp.int32)])
    mcol = jnp.concatenate([meta_edge_index[1], mar, jnp.zeros((pad,), jnp.int32)])
    mew = jnp.concatenate([meta_edge_weight, jnp.ones((NMET,), f32),
                           jnp.zeros((pad,), f32)])
    wdp = _meta_adj_kernel(mrow, mcol, mew)
    onehot = (product_idx[:, None] == mar[None, :]).astype(f32)
    prod = _tc_call(_meta_body, jax.ShapeDtypeStruct((B, H), f32))(
        wdp.reshape(NW, NMET, NMET), mg_mx, onehot,
        p['sg_meta_0_W'], r1(p['sg_meta_0_b']),
        p['sg_meta_1_W'], r1(p['sg_meta_1_b']),
        p['product_mlp_W1'], r1(p['product_mlp_b1']),
        r1(p['product_mlp_bn1_g']), r1(p['product_mlp_bn1_b']),
        p['product_mlp_W2'], r1(p['product_mlp_b2']),
        r1(p['product_mlp_bn2_g']), r1(p['product_mlp_bn2_b']))

    pg = _pert_gather_kernel(p['pert_emb'], pert_index.reshape(512))
    pert = _tc_call(_pert_body, jax.ShapeDtypeStruct((B, H), f32))(
        pg,
        p['pert_mlp_W1'], r1(p['pert_mlp_b1']),
        r1(p['pert_mlp_bn1_g']), r1(p['pert_mlp_bn1_b']),
        p['pert_mlp_W2'], r1(p['pert_mlp_b2']),
        r1(p['pert_mlp_bn2_g']), r1(p['pert_mlp_bn2_b']))

    wo = jnp.zeros((128, 256), f32).at[:2].set(p['fc_out_W'])
    bo = jnp.full((1, 128), -1e30, f32).at[0, :2].set(p['fc_out_b'])
    out = _tc_call(_final_body, jax.ShapeDtypeStruct((B, 128), f32))(
        fl3, pert, prod,
        p['ff1_W'], r1(p['ff1_b']), p['ff2_W'], r1(p['ff2_b']),
        p['ff3_W'], r1(p['ff3_b']), p['ff4_W'], r1(p['ff4_b']),
        wo, bo)
    return out[:, :2]


# speedup vs baseline: 5.8085x; 1.0253x over previous
import functools
import math

import jax
import jax.numpy as jnp
from jax import lax
from jax.experimental import pallas as pl
from jax.experimental.pallas import tpu as pltpu
from jax.experimental.pallas import tpu_sc as plsc

NC, NS, L = 2, 16, 16
NW = NC * NS
N = 8192
H = 128
NMET = 128
B = 64
E2 = 524288 + N
EPW = E2 // NW
NPB = N // NW
CH = 256
TOT = E2 + NW * 256 + NW * NW * 16
EM2P = 2560
EMW = EM2P // NW
INV = 1.0 / math.sqrt(1.0 + 1e-5)

_mesh = plsc.VectorSubcoreMesh(core_axis_name="c", subcore_axis_name="s",
                               num_cores=NC, num_subcores=NS)
_scp = pltpu.CompilerParams(needs_layout_passes=False)


def _sget(ref, idx):
    return ref[pl.ds(idx, L)][0]


@functools.partial(
    pl.kernel, mesh=_mesh, compiler_params=_scp,
    out_type=jax.ShapeDtypeStruct((NW, N), jnp.float32),
    scratch_types=[
        pltpu.VMEM((EPW,), jnp.int32),
        pltpu.VMEM((EPW,), jnp.float32),
        pltpu.VMEM((N,), jnp.float32),
    ])
def _deg_kernel(col_hbm, ew_hbm, out_hbm, col_v, ew_v, acc_v):
    c = lax.axis_index("c")
    s = lax.axis_index("s")
    wid = c * NS + s
    base = wid * EPW
    pltpu.sync_copy(col_hbm.at[pl.ds(base, EPW)], col_v)
    pltpu.sync_copy(ew_hbm.at[pl.ds(base, EPW)], ew_v)
    zero = jnp.zeros((L,), jnp.float32)

    def zb(i, carry):
        acc_v[pl.ds(i * L, L)] = zero
        return carry

    lax.fori_loop(0, N // L, zb, 0)

    def body(i, carry):
        cc = col_v[pl.ds(i * L, L)]
        w = ew_v[pl.ds(i * L, L)]
        plsc.addupdate_scatter(acc_v, [cc], w)
        return carry

    lax.fori_loop(0, EPW // L, body, 0)
    pltpu.sync_copy(acc_v, out_hbm.at[wid])


@functools.partial(
    pl.kernel, mesh=_mesh, compiler_params=_scp,
    out_type=(jax.ShapeDtypeStruct((E2,), jnp.float32),
              jax.ShapeDtypeStruct((NW, NW), jnp.float32)),
    scratch_types=[
        pltpu.VMEM((EPW,), jnp.int32),
        pltpu.VMEM((EPW,), jnp.int32),
        pltpu.VMEM((EPW,), jnp.float32),
        pltpu.VMEM((EPW,), jnp.float32),
        pltpu.VMEM((N,), jnp.float32),
        pltpu.VMEM((NW,), jnp.float32),
    ])
def _norm_kernel(row_hbm, col_hbm, ew_hbm, dis_hbm, out_hbm, cnt_hbm,
                 row_v, col_v, ew_v, nrm_v, dis_v, hist_v):
    c = lax.axis_index("c")
    s = lax.axis_index("s")
    wid = c * NS + s
    base = wid * EPW
    pltpu.sync_copy(row_hbm.at[pl.ds(base, EPW)], row_v)
    pltpu.sync_copy(col_hbm.at[pl.ds(base, EPW)], col_v)
    pltpu.sync_copy(ew_hbm.at[pl.ds(base, EPW)], ew_v)
    pltpu.sync_copy(dis_hbm, dis_v)
    zero = jnp.zeros((L,), jnp.float32)
    hist_v[pl.ds(0, L)] = zero
    hist_v[pl.ds(L, L)] = zero
    ones = jnp.ones((L,), jnp.float32)

    def body(i, carry):
        r = row_v[pl.ds(i * L, L)]
        cc = col_v[pl.ds(i * L, L)]
        w = ew_v[pl.ds(i * L, L)]
        dr = plsc.load_gather(dis_v, [r])
        dc = plsc.load_gather(dis_v, [cc])
        nrm_v[pl.ds(i * L, L)] = dr * w * dc
        bkt = lax.shift_right_logical(cc, 8)
        plsc.addupdate_scatter(hist_v, [bkt], ones)
        return carry

    lax.fori_loop(0, EPW // L, body, 0, unroll=2)
    pltpu.sync_copy(nrm_v, out_hbm.at[pl.ds(base, EPW)])
    pltpu.sync_copy(hist_v, cnt_hbm.at[wid])


@functools.partial(
    pl.kernel, mesh=_mesh, compiler_params=_scp,
    out_type=(jax.ShapeDtypeStruct((TOT,), jnp.int32),
              jax.ShapeDtypeStruct((TOT,), jnp.float32)),
    scratch_types=[
        pltpu.VMEM((EPW,), jnp.int32),
        pltpu.VMEM((EPW,), jnp.int32),
        pltpu.VMEM((EPW,), jnp.float32),
        pltpu.VMEM((EPW + L,), jnp.int32),
        pltpu.VMEM((EPW + L,), jnp.float32),
        pltpu.VMEM((NW * NW + L,), jnp.int32),
        pltpu.VMEM((NW + L,), jnp.int32),
        pltpu.VMEM((NW + L,), jnp.int32),
        pltpu.VMEM((L,), jnp.int32),
        pltpu.VMEM((L,), jnp.float32),
    ])
def _bucket_kernel(row_hbm, col_hbm, nrm_hbm, offs_hbm, tzs_hbm, tzn_hbm,
                   pk_hbm, nm_hbm,
                   row_v, col_v, nrm_v, pst_v, nst_v, offs_v, tzs_v, tzn_v,
                   zi_v, zf_v):
    c = lax.axis_index("c")
    s = lax.axis_index("s")
    wid = c * NS + s
    base = wid * EPW
    pltpu.sync_copy(row_hbm.at[pl.ds(base, EPW)], row_v)
    pltpu.sync_copy(col_hbm.at[pl.ds(base, EPW)], col_v)
    pltpu.sync_copy(nrm_hbm.at[pl.ds(base, EPW)], nrm_v)
    pltpu.sync_copy(offs_hbm, offs_v)
    pltpu.sync_copy(tzs_hbm, tzs_v)
    pltpu.sync_copy(tzn_hbm, tzn_v)
    zi_v[...] = jnp.zeros((L,), jnp.int32)
    zf_v[...] = jnp.zeros((L,), jnp.float32)

    for b in range(NW):
        def body(i, cnt):
            cc = col_v[pl.ds(i * L, L)]
            r = row_v[pl.ds(i * L, L)]
            w = nrm_v[pl.ds(i * L, L)]
            m = lax.shift_right_logical(cc, 8) == b
            pk = lax.bitwise_or(
                lax.shift_left(lax.bitwise_and(cc, NPB - 1), 13), r)
            plsc.store_compressed(pst_v.at[pl.ds(cnt, L)], pk, mask=m)
            plsc.store_compressed(nst_v.at[pl.ds(cnt, L)], w, mask=m)
            pc = plsc.all_reduce_population_count(m)
            return cnt + lax.reduce_max(pc, (0,))

        cnt = lax.fori_loop(0, EPW // L, body, 0)
        pst_v[pl.ds(cnt, L)] = jnp.zeros((L,), jnp.int32)
        nst_v[pl.ds(cnt, L)] = jnp.zeros((L,), jnp.float32)
        off = pl.multiple_of(_sget(offs_v, wid * NW + b), 16)
        n16 = (cnt + 15) // 16

        def wr(k, carry):
            pltpu.sync_copy(pst_v.at[pl.ds(k * L, L)],
                            pk_hbm.at[pl.ds(off + k * L, L)])
            pltpu.sync_copy(nst_v.at[pl.ds(k * L, L)],
                            nm_hbm.at[pl.ds(off + k * L, L)])
            return carry

        lax.fori_loop(0, n16, wr, 0)

    tz = pl.multiple_of(_sget(tzs_v, wid), 16)
    tn = _sget(tzn_v, wid)

    def tzb(k, carry):
        pltpu.sync_copy(zi_v, pk_hbm.at[pl.ds(tz + k * L, L)])
        pltpu.sync_copy(zf_v, nm_hbm.at[pl.ds(tz + k * L, L)])
        return carry

    lax.fori_loop(0, tn, tzb, 0)


@functools.partial(
    pl.kernel, mesh=_mesh, compiler_params=_scp,
    out_type=jax.ShapeDtypeStruct((N, H), jnp.float32),
    scratch_types=[
        pltpu.VMEM((NPB, H), jnp.float32),
        pltpu.VMEM((CH, H), jnp.float32),
        pltpu.VMEM((CH, H), jnp.float32),
        pltpu.VMEM((CH,), jnp.int32),
        pltpu.VMEM((CH,), jnp.int32),
        pltpu.VMEM((2, 128), jnp.int32),
        pltpu.VMEM((2, 128), jnp.int32),
        pltpu.VMEM((CH,), jnp.int32),
        pltpu.VMEM((CH,), jnp.int32),
        pltpu.VMEM((CH,), jnp.float32),
        pltpu.VMEM((CH,), jnp.float32),
        pltpu.VMEM((NW + L,), jnp.int32),
        pltpu.VMEM((NW + L,), jnp.int32),
        pltpu.SemaphoreType.DMA,
        pltpu.SemaphoreType.DMA,
    ])
def _hop_kernel(x_hbm, pk_hbm, nm_hbm, bases_hbm, nch_hbm, out_hbm,
                acc_v, rows0_v, rows1_v, pk0_v, pk1_v, gr0_v, gr1_v,
                lx0_v, lx1_v, nm0_v, nm1_v, bas_v, nch_v, sem0, sem1):
    c = lax.axis_index("c")
    s = lax.axis_index("s")
    wid = c * NS + s
    pltpu.sync_copy(bases_hbm, bas_v)
    pltpu.sync_copy(nch_hbm, nch_v)
    zero = jnp.zeros((L,), jnp.float32)

    def zb(i, carry):
        for q in range(H // L):
            acc_v[i, pl.ds(q * L, L)] = zero
        return carry

    lax.fori_loop(0, NPB, zb, 0)
    base = pl.multiple_of(_sget(bas_v, wid), 256)
    nch = _sget(nch_v, wid)

    bufs = ((rows0_v, pk0_v, gr0_v, lx0_v, nm0_v, sem0),
            (rows1_v, pk1_v, gr1_v, lx1_v, nm1_v, sem1))

    def stage(par, ch):
        rows_v, pk_v, gr_v, lx_v, nm_v, sem = bufs[par]
        pos = pl.multiple_of(base + ch * CH, 256)
        pltpu.sync_copy(pk_hbm.at[pl.ds(pos, CH)], pk_v)
        pltpu.sync_copy(nm_hbm.at[pl.ds(pos, CH)], nm_v)
        for k in range(CH // L):
            pk = pk_v[pl.ds(k * L, L)]
            gr_v[k // 8, pl.ds((k % 8) * L, L)] = lax.bitwise_and(pk, N - 1)
            lx_v[pl.ds(k * L, L)] = lax.shift_right_logical(pk, 13)
        d0 = pltpu.async_copy(x_hbm.at[gr_v.at[0]], rows_v.at[pl.ds(0, 128)], sem)
        d1 = pltpu.async_copy(x_hbm.at[gr_v.at[1]], rows_v.at[pl.ds(128, 128)], sem)
        return d0, d1

    def drain_accum(par):
        rows_v, pk_v, gr_v, lx_v, nm_v, sem = bufs[par]
        pltpu.make_async_copy(x_hbm.at[gr_v.at[0]], rows_v.at[pl.ds(0, 128)], sem).wait()
        pltpu.make_async_copy(x_hbm.at[gr_v.at[1]], rows_v.at[pl.ds(128, 128)], sem).wait()

        def ac(g, carry2):
            lv = lx_v[pl.ds(g * L, L)]
            wv = nm_v[pl.ds(g * L, L)]
            for j in range(L):
                i = g * L + j
                li = lv[j]
                w = wv[j]
                for q in range(H // L):
                    acc_v[li, pl.ds(q * L, L)] = (
                        acc_v[li, pl.ds(q * L, L)]
                        + rows_v[i, pl.ds(q * L, L)] * w)
            return carry2

        lax.fori_loop(0, CH // L, ac, 0)

    stage(0, 0)

    def pair(t, carry):
        stage(1, 2 * t + 1)
        drain_accum(0)

        @pl.when(2 * t + 2 < nch)
        def _():
            stage(0, 2 * t + 2)
        drain_accum(1)
        return carry

    npair = nch // 2

    def tail(_t, carry):
        return carry

    lax.fori_loop(0, npair, pair, 0)

    @pl.when(lax.rem(nch, 2) == 1)
    def _():
        @pl.when(nch > 1)
        def _():
            pass
        drain_accum(0)

    pltpu.sync_copy(acc_v, out_hbm.at[pl.ds(wid * NPB, NPB)])


@functools.partial(
    pl.kernel, mesh=_mesh, compiler_params=_scp,
    out_type=jax.ShapeDtypeStruct((N, H), jnp.float32),
    scratch_types=[
        pltpu.VMEM((NPB, H), jnp.float32),
        pltpu.VMEM((NMET, H), jnp.float32),
        pltpu.VMEM((CH,), jnp.int32),
        pltpu.VMEM((CH,), jnp.float32),
        pltpu.VMEM((NW + L,), jnp.int32),
        pltpu.VMEM((NW + L,), jnp.int32),
    ])
def _hop1_kernel(tab_hbm, pk_hbm, nm_hbm, bases_hbm, nch_hbm, out_hbm,
                 acc_v, tab_v, pk_v, nm_v, bas_v, nch_v):
    c = lax.axis_index("c")
    s = lax.axis_index("s")
    wid = c * NS + s
    pltpu.sync_copy(bases_hbm, bas_v)
    pltpu.sync_copy(nch_hbm, nch_v)
    pltpu.sync_copy(tab_hbm, tab_v)
    zero = jnp.zeros((L,), jnp.float32)

    def zb(i, carry):
        for q in range(H // L):
            acc_v[i, pl.ds(q * L, L)] = zero
        return carry

    lax.fori_loop(0, NPB, zb, 0)
    base = pl.multiple_of(_sget(bas_v, wid), 256)
    nch = _sget(nch_v, wid)

    def chunk(ch, carry):
        pos = pl.multiple_of(base + ch * CH, 256)
        pltpu.sync_copy(pk_hbm.at[pl.ds(pos, CH)], pk_v)
        pltpu.sync_copy(nm_hbm.at[pl.ds(pos, CH)], nm_v)

        def ac(g, carry2):
            pv = pk_v[pl.ds(g * L, L)]
            wv = nm_v[pl.ds(g * L, L)]
            lv = lax.shift_right_logical(pv, 13)
            rv = lax.bitwise_and(pv, NMET - 1)
            for j in range(L):
                li = lv[j]
                r = rv[j]
                w = wv[j]
                for q in range(H // L):
                    acc_v[li, pl.ds(q * L, L)] = (
                        acc_v[li, pl.ds(q * L, L)]
                        + tab_v[r, pl.ds(q * L, L)] * w)
            return carry2

        lax.fori_loop(0, CH // L, ac, 0)
        return carry

    lax.fori_loop(0, nch, chunk, 0)
    pltpu.sync_copy(acc_v, out_hbm.at[pl.ds(wid * NPB, NPB)])

@functools.partial(
    pl.kernel, mesh=_mesh, compiler_params=_scp,
    out_type=jax.ShapeDtypeStruct((NW, NMET * NMET), jnp.float32),
    scratch_types=[
        pltpu.VMEM((EMW,), jnp.int32),
        pltpu.VMEM((EMW,), jnp.int32),
        pltpu.VMEM((EMW,), jnp.float32),
        pltpu.VMEM((NMET * NMET,), jnp.float32),
    ])
def _meta_adj_kernel(row_hbm, col_hbm, ew_hbm, out_hbm,
                     row_v, col_v, ew_v, acc_v):
    c = lax.axis_index("c")
    s = lax.axis_index("s")
    wid = c * NS + s
    base = wid * EMW
    pltpu.sync_copy(row_hbm.at[pl.ds(base, EMW)], row_v)
    pltpu.sync_copy(col_hbm.at[pl.ds(base, EMW)], col_v)
    pltpu.sync_copy(ew_hbm.at[pl.ds(base, EMW)], ew_v)
    zero = jnp.zeros((L,), jnp.float32)

    def zb(i, carry):
        acc_v[pl.ds(i * L, L)] = zero
        return carry

    lax.fori_loop(0, NMET * NMET // L, zb, 0)

    def body(i, carry):
        r = row_v[pl.ds(i * L, L)]
        cc = col_v[pl.ds(i * L, L)]
        w = ew_v[pl.ds(i * L, L)]
        flat = cc * NMET + r
        plsc.addupdate_scatter(acc_v, [flat], w)
        return carry

    lax.fori_loop(0, EMW // L, body, 0)
    pltpu.sync_copy(acc_v, out_hbm.at[wid])


@functools.partial(
    pl.kernel, mesh=_mesh, compiler_params=_scp,
    out_type=jax.ShapeDtypeStruct((512, H), jnp.float32),
    scratch_types=[
        pltpu.VMEM((16,), jnp.int32),
        pltpu.VMEM((16, H), jnp.float32),
        pltpu.SemaphoreType.DMA,
    ])
def _pert_gather_kernel(tab_hbm, idx_hbm, out_hbm, idx_v, rows_v, sem):
    c = lax.axis_index("c")
    s = lax.axis_index("s")
    wid = c * NS + s
    pltpu.sync_copy(idx_hbm.at[pl.ds(wid * 16, 16)], idx_v)
    pltpu.async_copy(tab_hbm.at[idx_v], rows_v, sem).wait()
    pltpu.sync_copy(rows_v, out_hbm.at[pl.ds(wid * 16, 16)])


def _tc_call(body, out_shape, grid=None, in_specs=None, out_specs=None):
    kw = {}
    if grid is not None:
        kw.update(grid=grid, in_specs=in_specs, out_specs=out_specs)
    return pl.pallas_call(body, out_shape=out_shape, **kw)


def _dis_body(d_ref, o_ref):
    d = jnp.sum(d_ref[...], axis=0)
    o_ref[...] = jnp.where(d > 0, lax.rsqrt(jnp.maximum(d, 1e-12)), 0.0)


def _maxnorm_body(x_ref, o_ref):
    x = x_ref[...]
    rn = jnp.sqrt(jnp.sum(x * x, axis=1, keepdims=True))
    o_ref[...] = jnp.where(rn > 1.0, x / jnp.maximum(rn, 1e-12), x)


def _dot_t(x, w):
    return lax.dot_general(x, w, (((1,), (1,)), ((), ())),
                           preferred_element_type=jnp.float32)


def _linear_relu_body(x_ref, w_ref, b_ref, o_ref):
    z = _dot_t(x_ref[...], w_ref[...]) + b_ref[...]
    o_ref[...] = jnp.maximum(z, 0.0)


def _sg_embmlp_body(x_ref, wsg_ref, bsg_ref, w1_ref, b1_ref, g1_ref,
                    t1_ref, w2_ref, b2_ref, g2_ref, t2_ref, o_ref):
    z = _dot_t(x_ref[...], wsg_ref[...]) + bsg_ref[...]
    h = _dot_t(z, w1_ref[...]) + b1_ref[...]
    h = jnp.maximum(h * (g1_ref[...] * INV) + t1_ref[...], 0.0)
    h2 = _dot_t(h, w2_ref[...]) + b2_ref[...]
    o_ref[...] = h2 * (g2_ref[...] * INV) + t2_ref[...]


def _flat1_body(x_ref, w_ref, b_ref, g_ref, t_ref, o_ref):
    k = pl.program_id(0)

    @pl.when(k == 0)
    def _():
        o_ref[...] = jnp.zeros_like(o_ref)

    o_ref[...] += _dot_t(x_ref[...], w_ref[...])

    @pl.when(k == pl.num_programs(0) - 1)
    def _():
        z = o_ref[...] + b_ref[...]
        o_ref[...] = jnp.maximum(z * (g_ref[...] * INV) + t_ref[...], 0.0)


def _flat23_body(x_ref, w2_ref, b2_ref, g2_ref, t2_ref, w3_ref, b3_ref,
                 g3_ref, t3_ref, o_ref):
    h = _dot_t(x_ref[...], w2_ref[...]) + b2_ref[...]
    h = jnp.maximum(h * (g2_ref[...] * INV) + t2_ref[...], 0.0)
    h2 = _dot_t(h, w3_ref[...]) + b3_ref[...]
    o_ref[...] = jnp.maximum(h2 * (g3_ref[...] * INV) + t3_ref[...], 0.0)


def _meta_body(wdp_ref, mg_ref, oh_ref, w0_ref, b0_ref, w1_ref, b1_ref,
               pw1_ref, pb1_ref, pg1_ref, pt1_ref, pw2_ref, pb2_ref,
               pg2_ref, pt2_ref, o_ref):
    wd = jnp.sum(wdp_ref[...], axis=0).reshape(NMET, NMET)
    deg = jnp.sum(wd, axis=1, keepdims=True)
    dis = jnp.where(deg > 0, lax.rsqrt(jnp.maximum(deg, 1e-12)), 0.0)
    dis2 = dis * dis

    def a2(v):
        t = dis * v
        t = lax.dot_general(wd, t, (((1,), (0,)), ((), ())),
                            preferred_element_type=jnp.float32)
        t = dis2 * t
        t = lax.dot_general(wd, t, (((1,), (0,)), ((), ())),
                            preferred_element_type=jnp.float32)
        return dis * t

    m = mg_ref[...]
    m = _dot_t(a2(m), w0_ref[...]) + b0_ref[...]
    m = jnp.maximum(m, 0.0)
    m = _dot_t(a2(m), w1_ref[...]) + b1_ref[...]
    pe = lax.dot_general(oh_ref[...], m, (((1,), (0,)), ((), ())),
                         preferred_element_type=jnp.float32)
    h = _dot_t(pe, pw1_ref[...]) + pb1_ref[...]
    h = jnp.maximum(h * (pg1_ref[...] * INV) + pt1_ref[...], 0.0)
    h2 = _dot_t(h, pw2_ref[...]) + pb2_ref[...]
    o_ref[...] = h2 * (pg2_ref[...] * INV) + pt2_ref[...]


def _pert_body(g_ref, pw1_ref, pb1_ref, pg1_ref, pt1_ref, pw2_ref, pb2_ref,
               pg2_ref, pt2_ref, o_ref):
    v = g_ref[...]
    rn = jnp.sqrt(jnp.sum(v * v, axis=1, keepdims=True))
    v = jnp.where(rn > 1.0, v / jnp.maximum(rn, 1e-12), v)
    r = lax.broadcasted_iota(jnp.int32, (B, 512), 0)
    k = lax.broadcasted_iota(jnp.int32, (B, 512), 1)
    sel = jnp.where(lax.div(k, 8) == r, 1.0, 0.0)
    sv = lax.dot_general(sel, v, (((1,), (0,)), ((), ())),
                         preferred_element_type=jnp.float32)
    h = _dot_t(sv, pw1_ref[...]) + pb1_ref[...]
    h = jnp.maximum(h * (pg1_ref[...] * INV) + pt1_ref[...], 0.0)
    h2 = _dot_t(h, pw2_ref[...]) + pb2_ref[...]
    o_ref[...] = h2 * (pg2_ref[...] * INV) + pt2_ref[...]


def _final_body(x_ref, pe_ref, pr_ref, w1_ref, b1_ref, w2_ref, b2_ref,
                w3_ref, b3_ref, w4_ref, b4_ref, wo_ref, bo_ref, o_ref):
    x = jnp.concatenate([x_ref[...], pe_ref[...], pr_ref[...]], axis=1)
    h = jnp.maximum(_dot_t(x, w1_ref[...]) + b1_ref[...], 0.0)
    h = jnp.maximum(_dot_t(h, w2_ref[...]) + b2_ref[...], 0.0)
    h = jnp.maximum(_dot_t(h, w3_ref[...]) + b3_ref[...], 0.0)
    h = _dot_t(h, w4_ref[...]) + b4_ref[...]
    z = _dot_t(h, wo_ref[...]) + bo_ref[...]
    m = jnp.max(z, axis=1, keepdims=True)
    e = jnp.exp(z - m)
    o_ref[...] = e / jnp.sum(e, axis=1, keepdims=True)


def kernel(edge_index, edge_weight, meta_edge_index, meta_edge_weight,
           product_idx, pert_index, batch, params):
    p = params
    f32 = jnp.float32

    ar = jnp.arange(N, dtype=jnp.int32)
    row = jnp.concatenate([edge_index[0], ar])
    col = jnp.concatenate([edge_index[1], ar])
    ew2 = jnp.concatenate([edge_weight, jnp.ones((N,), f32)])

    degp = _deg_kernel(col, ew2)
    dis64 = _tc_call(_dis_body, jax.ShapeDtypeStruct((N // 128, 128), f32))(
        degp.reshape(NW, N // 128, 128))
    dis = dis64.reshape(N)
    norm, cntf = _norm_kernel(row, col, ew2, dis)

    cnts = cntf.astype(jnp.int32)
    c16 = ((cnts + 15) // 16) * 16
    tot16 = jnp.sum(c16, axis=0)
    caps = ((tot16 + 255) // 256) * 256
    bases = jnp.concatenate([jnp.zeros((1,), jnp.int32),
                             jnp.cumsum(caps)[:-1].astype(jnp.int32)])
    excl = jnp.concatenate([jnp.zeros((1, NW), jnp.int32),
                            jnp.cumsum(c16, axis=0)[:-1].astype(jnp.int32)],
                           axis=0)
    padL = jnp.zeros((L,), jnp.int32)
    offs = jnp.concatenate([(bases[None, :] + excl).reshape(-1), padL])
    tzs = jnp.concatenate([bases + tot16, padL])
    tzn = jnp.concatenate([(caps - tot16) // 16, padL])
    nch = jnp.concatenate([caps // CH, padL])
    bases_p = jnp.concatenate([bases, padL])

    pk, nm = _bucket_kernel(row, col, norm, offs, tzs, tzn)

    mg_mx = _tc_call(_maxnorm_body, jax.ShapeDtypeStruct((NMET, H), f32))(
        p['meta_graph_emb'])
    def r1(v):
        return v.reshape(1, -1)

    h1 = _hop1_kernel(mg_mx, pk, nm, bases_p, nch)
    h2 = _hop_kernel(h1, pk, nm, bases_p, nch)
    grid16 = (16,)
    bs_x = pl.BlockSpec((512, H), lambda i: (i, 0))
    bs_w = pl.BlockSpec((H, H), lambda i: (0, 0))
    bs_b = pl.BlockSpec((1, H), lambda i: (0, 0))
    x1 = _tc_call(_linear_relu_body, jax.ShapeDtypeStruct((N, H), f32),
                  grid=grid16, in_specs=[bs_x, bs_w, bs_b],
                  out_specs=bs_x)(h2, p['sg_gem_0_W'], r1(p['sg_gem_0_b']))
    h3 = _hop_kernel(x1, pk, nm, bases_p, nch)
    h4 = _hop_kernel(h3, pk, nm, bases_p, nch)
    base_emb = _tc_call(
        _sg_embmlp_body, jax.ShapeDtypeStruct((N, H), f32), grid=grid16,
        in_specs=[bs_x, bs_w, bs_b] + [bs_w, bs_b, bs_b, bs_b] * 2,
        out_specs=bs_x)(
        h4, p['sg_gem_1_W'], r1(p['sg_gem_1_b']),
        p['emb_mlp_W1'], r1(p['emb_mlp_b1']), r1(p['emb_mlp_bn1_g']), r1(p['emb_mlp_bn1_b']),
        p['emb_mlp_W2'], r1(p['emb_mlp_b2']), r1(p['emb_mlp_bn2_g']), r1(p['emb_mlp_bn2_b']))

    xflat = base_emb.reshape(B, NMET * H)
    fl1 = _tc_call(
        _flat1_body, jax.ShapeDtypeStruct((B, 1024), f32), grid=(32,),
        in_specs=[pl.BlockSpec((B, 512), lambda k: (0, k)),
                  pl.BlockSpec((1024, 512), lambda k: (0, k)),
                  pl.BlockSpec((1, 1024), lambda k: (0, 0)),
                  pl.BlockSpec((1, 1024), lambda k: (0, 0)),
                  pl.BlockSpec((1, 1024), lambda k: (0, 0))],
        out_specs=pl.BlockSpec((B, 1024), lambda k: (0, 0)))(
        xflat, p['flat_fc1_W'], r1(p['flat_fc1_b']),
        r1(p['flat_bn1_g']), r1(p['flat_bn1_b']))
    fl3 = _tc_call(_flat23_body, jax.ShapeDtypeStruct((B, H), f32))(
        fl1, p['flat_fc2_W'], r1(p['flat_fc2_b']), r1(p['flat_bn2_g']), r1(p['flat_bn2_b']),
        p['flat_fc3_W'], r1(p['flat_fc3_b']), r1(p['flat_bn3_g']), r1(p['flat_bn3_b']))

    mar = jnp.arange(NMET, dtype=jnp.int32)
    pad = EM2P - (meta_edge_index.shape[1] + NMET)
    mrow = jnp.concatenate([meta_edge_index[0], mar, jnp.zeros((pad,), jnp.int32)])
    mcol = jnp.concatenate([meta_edge_index[1], mar, jnp.zeros((pad,), jnp.int32)])
    mew = jnp.concatenate([meta_edge_weight, jnp.ones((NMET,), f32),
                           jnp.zeros((pad,), f32)])
    wdp = _meta_adj_kernel(mrow, mcol, mew)
    onehot = (product_idx[:, None] == mar[None, :]).astype(f32)
    prod = _tc_call(_meta_body, jax.ShapeDtypeStruct((B, H), f32))(
        wdp.reshape(NW, NMET, NMET), mg_mx, onehot,
        p['sg_meta_0_W'], r1(p['sg_meta_0_b']),
        p['sg_meta_1_W'], r1(p['sg_meta_1_b']),
        p['product_mlp_W1'], r1(p['product_mlp_b1']),
        r1(p['product_mlp_bn1_g']), r1(p['product_mlp_bn1_b']),
        p['product_mlp_W2'], r1(p['product_mlp_b2']),
        r1(p['product_mlp_bn2_g']), r1(p['product_mlp_bn2_b']))

    pg = _pert_gather_kernel(p['pert_emb'], pert_index.reshape(512))
    pert = _tc_call(_pert_body, jax.ShapeDtypeStruct((B, H), f32))(
        pg,
        p['pert_mlp_W1'], r1(p['pert_mlp_b1']),
        r1(p['pert_mlp_bn1_g']), r1(p['pert_mlp_bn1_b']),
        p['pert_mlp_W2'], r1(p['pert_mlp_b2']),
        r1(p['pert_mlp_bn2_g']), r1(p['pert_mlp_bn2_b']))

    wo = jnp.zeros((128, 256), f32).at[:2].set(p['fc_out_W'])
    bo = jnp.full((1, 128), -1e30, f32).at[0, :2].set(p['fc_out_b'])
    out = _tc_call(_final_body, jax.ShapeDtypeStruct((B, 128), f32))(
        fl3, pert, prod,
        p['ff1_W'], r1(p['ff1_b']), p['ff2_W'], r1(p['ff2_b']),
        p['ff3_W'], r1(p['ff3_b']), p['ff4_W'], r1(p['ff4_b']),
        wo, bo)
    return out[:, :2]


# speedup vs baseline: 5.8108x; 1.0004x over previous
import functools
import math

import jax
import jax.numpy as jnp
from jax import lax
from jax.experimental import pallas as pl
from jax.experimental.pallas import tpu as pltpu
from jax.experimental.pallas import tpu_sc as plsc

NC, NS, L = 2, 16, 16
NW = NC * NS
N = 8192
H = 128
NMET = 128
B = 64
E2 = 524288 + N
EPW = E2 // NW
NPB = N // NW
CH = 256
TOT = E2 + NW * 256 + NW * NW * 16
EM2P = 2560
EMW = EM2P // NW
INV = 1.0 / math.sqrt(1.0 + 1e-5)

_mesh = plsc.VectorSubcoreMesh(core_axis_name="c", subcore_axis_name="s",
                               num_cores=NC, num_subcores=NS)
_scp = pltpu.CompilerParams(needs_layout_passes=False)


def _sget(ref, idx):
    return ref[pl.ds(idx, L)][0]


@functools.partial(
    pl.kernel, mesh=_mesh, compiler_params=_scp,
    out_type=jax.ShapeDtypeStruct((NW, N), jnp.float32),
    scratch_types=[
        pltpu.VMEM((EPW,), jnp.int32),
        pltpu.VMEM((EPW,), jnp.float32),
        pltpu.VMEM((N,), jnp.float32),
    ])
def _deg_kernel(col_hbm, ew_hbm, out_hbm, col_v, ew_v, acc_v):
    c = lax.axis_index("c")
    s = lax.axis_index("s")
    wid = c * NS + s
    base = wid * EPW
    pltpu.sync_copy(col_hbm.at[pl.ds(base, EPW)], col_v)
    pltpu.sync_copy(ew_hbm.at[pl.ds(base, EPW)], ew_v)
    zero = jnp.zeros((L,), jnp.float32)

    def zb(i, carry):
        acc_v[pl.ds(i * L, L)] = zero
        return carry

    lax.fori_loop(0, N // L, zb, 0)

    def body(i, carry):
        cc = col_v[pl.ds(i * L, L)]
        w = ew_v[pl.ds(i * L, L)]
        plsc.addupdate_scatter(acc_v, [cc], w)
        return carry

    lax.fori_loop(0, EPW // L, body, 0)
    pltpu.sync_copy(acc_v, out_hbm.at[wid])


@functools.partial(
    pl.kernel, mesh=_mesh, compiler_params=_scp,
    out_type=(jax.ShapeDtypeStruct((E2,), jnp.float32),
              jax.ShapeDtypeStruct((NW, NW), jnp.float32)),
    scratch_types=[
        pltpu.VMEM((EPW,), jnp.int32),
        pltpu.VMEM((EPW,), jnp.int32),
        pltpu.VMEM((EPW,), jnp.float32),
        pltpu.VMEM((EPW,), jnp.float32),
        pltpu.VMEM((N,), jnp.float32),
        pltpu.VMEM((NW,), jnp.float32),
    ])
def _norm_kernel(row_hbm, col_hbm, ew_hbm, dis_hbm, out_hbm, cnt_hbm,
                 row_v, col_v, ew_v, nrm_v, dis_v, hist_v):
    c = lax.axis_index("c")
    s = lax.axis_index("s")
    wid = c * NS + s
    base = wid * EPW
    pltpu.sync_copy(row_hbm.at[pl.ds(base, EPW)], row_v)
    pltpu.sync_copy(col_hbm.at[pl.ds(base, EPW)], col_v)
    pltpu.sync_copy(ew_hbm.at[pl.ds(base, EPW)], ew_v)
    pltpu.sync_copy(dis_hbm, dis_v)
    zero = jnp.zeros((L,), jnp.float32)
    hist_v[pl.ds(0, L)] = zero
    hist_v[pl.ds(L, L)] = zero
    ones = jnp.ones((L,), jnp.float32)

    def body(i, carry):
        r = row_v[pl.ds(i * L, L)]
        cc = col_v[pl.ds(i * L, L)]
        w = ew_v[pl.ds(i * L, L)]
        dr = plsc.load_gather(dis_v, [r])
        dc = plsc.load_gather(dis_v, [cc])
        nrm_v[pl.ds(i * L, L)] = dr * w * dc
        bkt = lax.shift_right_logical(cc, 8)
        plsc.addupdate_scatter(hist_v, [bkt], ones)
        return carry

    lax.fori_loop(0, EPW // L, body, 0, unroll=2)
    pltpu.sync_copy(nrm_v, out_hbm.at[pl.ds(base, EPW)])
    pltpu.sync_copy(hist_v, cnt_hbm.at[wid])


@functools.partial(
    pl.kernel, mesh=_mesh, compiler_params=_scp,
    out_type=(jax.ShapeDtypeStruct((TOT,), jnp.int32),
              jax.ShapeDtypeStruct((TOT,), jnp.float32)),
    scratch_types=[
        pltpu.VMEM((EPW,), jnp.int32),
        pltpu.VMEM((EPW,), jnp.int32),
        pltpu.VMEM((EPW,), jnp.float32),
        pltpu.VMEM((EPW + L,), jnp.int32),
        pltpu.VMEM((EPW + L,), jnp.float32),
        pltpu.VMEM((NW * NW + L,), jnp.int32),
        pltpu.VMEM((NW + L,), jnp.int32),
        pltpu.VMEM((NW + L,), jnp.int32),
        pltpu.VMEM((L,), jnp.int32),
        pltpu.VMEM((L,), jnp.float32),
    ])
def _bucket_kernel(row_hbm, col_hbm, nrm_hbm, offs_hbm, tzs_hbm, tzn_hbm,
                   pk_hbm, nm_hbm,
                   row_v, col_v, nrm_v, pst_v, nst_v, offs_v, tzs_v, tzn_v,
                   zi_v, zf_v):
    c = lax.axis_index("c")
    s = lax.axis_index("s")
    wid = c * NS + s
    base = wid * EPW
    pltpu.sync_copy(row_hbm.at[pl.ds(base, EPW)], row_v)
    pltpu.sync_copy(col_hbm.at[pl.ds(base, EPW)], col_v)
    pltpu.sync_copy(nrm_hbm.at[pl.ds(base, EPW)], nrm_v)
    pltpu.sync_copy(offs_hbm, offs_v)
    pltpu.sync_copy(tzs_hbm, tzs_v)
    pltpu.sync_copy(tzn_hbm, tzn_v)
    zi_v[...] = jnp.zeros((L,), jnp.int32)
    zf_v[...] = jnp.zeros((L,), jnp.float32)

    for b in range(NW):
        def body(i, cnt):
            cc = col_v[pl.ds(i * L, L)]
            r = row_v[pl.ds(i * L, L)]
            w = nrm_v[pl.ds(i * L, L)]
            m = lax.shift_right_logical(cc, 8) == b
            pk = lax.bitwise_or(
                lax.shift_left(lax.bitwise_and(cc, NPB - 1), 13), r)
            plsc.store_compressed(pst_v.at[pl.ds(cnt, L)], pk, mask=m)
            plsc.store_compressed(nst_v.at[pl.ds(cnt, L)], w, mask=m)
            pc = plsc.all_reduce_population_count(m)
            return cnt + lax.reduce_max(pc, (0,))

        cnt = lax.fori_loop(0, EPW // L, body, 0)
        pst_v[pl.ds(cnt, L)] = jnp.zeros((L,), jnp.int32)
        nst_v[pl.ds(cnt, L)] = jnp.zeros((L,), jnp.float32)
        off = pl.multiple_of(_sget(offs_v, wid * NW + b), 16)
        n16 = (cnt + 15) // 16

        def wr(k, carry):
            pltpu.sync_copy(pst_v.at[pl.ds(k * L, L)],
                            pk_hbm.at[pl.ds(off + k * L, L)])
            pltpu.sync_copy(nst_v.at[pl.ds(k * L, L)],
                            nm_hbm.at[pl.ds(off + k * L, L)])
            return carry

        lax.fori_loop(0, n16, wr, 0)

    tz = pl.multiple_of(_sget(tzs_v, wid), 16)
    tn = _sget(tzn_v, wid)

    def tzb(k, carry):
        pltpu.sync_copy(zi_v, pk_hbm.at[pl.ds(tz + k * L, L)])
        pltpu.sync_copy(zf_v, nm_hbm.at[pl.ds(tz + k * L, L)])
        return carry

    lax.fori_loop(0, tn, tzb, 0)


@functools.partial(
    pl.kernel, mesh=_mesh, compiler_params=_scp,
    out_type=jax.ShapeDtypeStruct((N, H), jnp.float32),
    scratch_types=[
        pltpu.VMEM((NPB, H), jnp.float32),
        pltpu.VMEM((CH, H), jnp.float32),
        pltpu.VMEM((CH, H), jnp.float32),
        pltpu.VMEM((CH,), jnp.int32),
        pltpu.VMEM((CH,), jnp.int32),
        pltpu.VMEM((2, 128), jnp.int32),
        pltpu.VMEM((2, 128), jnp.int32),
        pltpu.VMEM((CH,), jnp.int32),
        pltpu.VMEM((CH,), jnp.int32),
        pltpu.VMEM((CH,), jnp.float32),
        pltpu.VMEM((CH,), jnp.float32),
        pltpu.VMEM((NW + L,), jnp.int32),
        pltpu.VMEM((NW + L,), jnp.int32),
        pltpu.SemaphoreType.DMA,
        pltpu.SemaphoreType.DMA,
    ])
def _hop_kernel(x_hbm, pk_hbm, nm_hbm, bases_hbm, nch_hbm, out_hbm,
                acc_v, rows0_v, rows1_v, pk0_v, pk1_v, gr0_v, gr1_v,
                lx0_v, lx1_v, nm0_v, nm1_v, bas_v, nch_v, sem0, sem1):
    c = lax.axis_index("c")
    s = lax.axis_index("s")
    wid = c * NS + s
    pltpu.sync_copy(bases_hbm, bas_v)
    pltpu.sync_copy(nch_hbm, nch_v)
    zero = jnp.zeros((L,), jnp.float32)

    def zb(i, carry):
        for q in range(H // L):
            acc_v[i, pl.ds(q * L, L)] = zero
        return carry

    lax.fori_loop(0, NPB, zb, 0)
    base = pl.multiple_of(_sget(bas_v, wid), 256)
    nch = _sget(nch_v, wid)

    bufs = ((rows0_v, pk0_v, gr0_v, lx0_v, nm0_v, sem0),
            (rows1_v, pk1_v, gr1_v, lx1_v, nm1_v, sem1))

    def stage(par, ch):
        rows_v, pk_v, gr_v, lx_v, nm_v, sem = bufs[par]
        pos = pl.multiple_of(base + ch * CH, 256)
        pltpu.sync_copy(pk_hbm.at[pl.ds(pos, CH)], pk_v)
        pltpu.sync_copy(nm_hbm.at[pl.ds(pos, CH)], nm_v)
        for k in range(CH // L):
            pk = pk_v[pl.ds(k * L, L)]
            gr_v[k // 8, pl.ds((k % 8) * L, L)] = lax.bitwise_and(pk, N - 1)
            lx_v[pl.ds(k * L, L)] = lax.shift_right_logical(pk, 13)
        d0 = pltpu.async_copy(x_hbm.at[gr_v.at[0]], rows_v.at[pl.ds(0, 128)], sem)
        d1 = pltpu.async_copy(x_hbm.at[gr_v.at[1]], rows_v.at[pl.ds(128, 128)], sem)
        return d0, d1

    def drain_accum(par):
        rows_v, pk_v, gr_v, lx_v, nm_v, sem = bufs[par]
        pltpu.make_async_copy(x_hbm.at[gr_v.at[0]], rows_v.at[pl.ds(0, 128)], sem).wait()
        pltpu.make_async_copy(x_hbm.at[gr_v.at[1]], rows_v.at[pl.ds(128, 128)], sem).wait()

        def ac(g, carry2):
            lv = lx_v[pl.ds(g * L, L)]
            wv = nm_v[pl.ds(g * L, L)]
            for j in range(L):
                i = g * L + j
                li = lv[j]
                w = wv[j]
                for q in range(H // L):
                    acc_v[li, pl.ds(q * L, L)] = (
                        acc_v[li, pl.ds(q * L, L)]
                        + rows_v[i, pl.ds(q * L, L)] * w)
            return carry2

        lax.fori_loop(0, CH // L, ac, 0)

    stage(0, 0)

    def pair(t, carry):
        stage(1, 2 * t + 1)
        drain_accum(0)

        @pl.when(2 * t + 2 < nch)
        def _():
            stage(0, 2 * t + 2)
        drain_accum(1)
        return carry

    lax.fori_loop(0, nch // 2, pair, 0)

    @pl.when(lax.rem(nch, 2) == 1)
    def _():
        drain_accum(0)

    pltpu.sync_copy(acc_v, out_hbm.at[pl.ds(wid * NPB, NPB)])


@functools.partial(
    pl.kernel, mesh=_mesh, compiler_params=_scp,
    out_type=jax.ShapeDtypeStruct((N, H), jnp.float32),
    scratch_types=[
        pltpu.VMEM((NPB, H), jnp.float32),
        pltpu.VMEM((NMET, H), jnp.float32),
        pltpu.VMEM((CH,), jnp.int32),
        pltpu.VMEM((CH,), jnp.float32),
        pltpu.VMEM((NW + L,), jnp.int32),
        pltpu.VMEM((NW + L,), jnp.int32),
    ])
def _hop1_kernel(tab_hbm, pk_hbm, nm_hbm, bases_hbm, nch_hbm, out_hbm,
                 acc_v, tab_v, pk_v, nm_v, bas_v, nch_v):
    c = lax.axis_index("c")
    s = lax.axis_index("s")
    wid = c * NS + s
    pltpu.sync_copy(bases_hbm, bas_v)
    pltpu.sync_copy(nch_hbm, nch_v)
    pltpu.sync_copy(tab_hbm, tab_v)
    zero = jnp.zeros((L,), jnp.float32)

    def zb(i, carry):
        for q in range(H // L):
            acc_v[i, pl.ds(q * L, L)] = zero
        return carry

    lax.fori_loop(0, NPB, zb, 0)
    base = pl.multiple_of(_sget(bas_v, wid), 256)
    nch = _sget(nch_v, wid)

    def chunk(ch, carry):
        pos = pl.multiple_of(base + ch * CH, 256)
        pltpu.sync_copy(pk_hbm.at[pl.ds(pos, CH)], pk_v)
        pltpu.sync_copy(nm_hbm.at[pl.ds(pos, CH)], nm_v)

        def ac(g, carry2):
            pv = pk_v[pl.ds(g * L, L)]
            wv = nm_v[pl.ds(g * L, L)]
            lv = lax.shift_right_logical(pv, 13)
            rv = lax.bitwise_and(pv, NMET - 1)
            for j in range(L):
                li = lv[j]
                r = rv[j]
                w = wv[j]
                for q in range(H // L):
                    acc_v[li, pl.ds(q * L, L)] = (
                        acc_v[li, pl.ds(q * L, L)]
                        + tab_v[r, pl.ds(q * L, L)] * w)
            return carry2

        lax.fori_loop(0, CH // L, ac, 0)
        return carry

    lax.fori_loop(0, nch, chunk, 0)
    pltpu.sync_copy(acc_v, out_hbm.at[pl.ds(wid * NPB, NPB)])

@functools.partial(
    pl.kernel, mesh=_mesh, compiler_params=_scp,
    out_type=jax.ShapeDtypeStruct((NW, NMET * NMET), jnp.float32),
    scratch_types=[
        pltpu.VMEM((EMW,), jnp.int32),
        pltpu.VMEM((EMW,), jnp.int32),
        pltpu.VMEM((EMW,), jnp.float32),
        pltpu.VMEM((NMET * NMET,), jnp.float32),
    ])
def _meta_adj_kernel(row_hbm, col_hbm, ew_hbm, out_hbm,
                     row_v, col_v, ew_v, acc_v):
    c = lax.axis_index("c")
    s = lax.axis_index("s")
    wid = c * NS + s
    base = wid * EMW
    pltpu.sync_copy(row_hbm.at[pl.ds(base, EMW)], row_v)
    pltpu.sync_copy(col_hbm.at[pl.ds(base, EMW)], col_v)
    pltpu.sync_copy(ew_hbm.at[pl.ds(base, EMW)], ew_v)
    zero = jnp.zeros((L,), jnp.float32)

    def zb(i, carry):
        acc_v[pl.ds(i * L, L)] = zero
        return carry

    lax.fori_loop(0, NMET * NMET // L, zb, 0)

    def body(i, carry):
        r = row_v[pl.ds(i * L, L)]
        cc = col_v[pl.ds(i * L, L)]
        w = ew_v[pl.ds(i * L, L)]
        flat = cc * NMET + r
        plsc.addupdate_scatter(acc_v, [flat], w)
        return carry

    lax.fori_loop(0, EMW // L, body, 0)
    pltpu.sync_copy(acc_v, out_hbm.at[wid])


@functools.partial(
    pl.kernel, mesh=_mesh, compiler_params=_scp,
    out_type=jax.ShapeDtypeStruct((512, H), jnp.float32),
    scratch_types=[
        pltpu.VMEM((16,), jnp.int32),
        pltpu.VMEM((16, H), jnp.float32),
        pltpu.SemaphoreType.DMA,
    ])
def _pert_gather_kernel(tab_hbm, idx_hbm, out_hbm, idx_v, rows_v, sem):
    c = lax.axis_index("c")
    s = lax.axis_index("s")
    wid = c * NS + s
    pltpu.sync_copy(idx_hbm.at[pl.ds(wid * 16, 16)], idx_v)
    pltpu.async_copy(tab_hbm.at[idx_v], rows_v, sem).wait()
    pltpu.sync_copy(rows_v, out_hbm.at[pl.ds(wid * 16, 16)])


def _tc_call(body, out_shape, grid=None, in_specs=None, out_specs=None):
    kw = {}
    if grid is not None:
        kw.update(grid=grid, in_specs=in_specs, out_specs=out_specs)
    return pl.pallas_call(body, out_shape=out_shape, **kw)


def _dis_body(d_ref, o_ref):
    d = jnp.sum(d_ref[...], axis=0)
    o_ref[...] = jnp.where(d > 0, lax.rsqrt(jnp.maximum(d, 1e-12)), 0.0)


def _maxnorm_body(x_ref, o_ref):
    x = x_ref[...]
    rn = jnp.sqrt(jnp.sum(x * x, axis=1, keepdims=True))
    o_ref[...] = jnp.where(rn > 1.0, x / jnp.maximum(rn, 1e-12), x)


def _dot_t(x, w):
    return lax.dot_general(x, w, (((1,), (1,)), ((), ())),
                           preferred_element_type=jnp.float32)


def _linear_relu_body(x_ref, w_ref, b_ref, o_ref):
    z = _dot_t(x_ref[...], w_ref[...]) + b_ref[...]
    o_ref[...] = jnp.maximum(z, 0.0)


def _sg_embmlp_body(x_ref, wsg_ref, bsg_ref, w1_ref, b1_ref, g1_ref,
                    t1_ref, w2_ref, b2_ref, g2_ref, t2_ref, o_ref):
    z = _dot_t(x_ref[...], wsg_ref[...]) + bsg_ref[...]
    h = _dot_t(z, w1_ref[...]) + b1_ref[...]
    h = jnp.maximum(h * (g1_ref[...] * INV) + t1_ref[...], 0.0)
    h2 = _dot_t(h, w2_ref[...]) + b2_ref[...]
    o_ref[...] = h2 * (g2_ref[...] * INV) + t2_ref[...]


def _flat1_body(x_ref, w_ref, b_ref, g_ref, t_ref, o_ref):
    k = pl.program_id(0)

    @pl.when(k == 0)
    def _():
        o_ref[...] = jnp.zeros_like(o_ref)

    o_ref[...] += _dot_t(x_ref[...], w_ref[...])

    @pl.when(k == pl.num_programs(0) - 1)
    def _():
        z = o_ref[...] + b_ref[...]
        o_ref[...] = jnp.maximum(z * (g_ref[...] * INV) + t_ref[...], 0.0)


def _flat23_body(x_ref, w2_ref, b2_ref, g2_ref, t2_ref, w3_ref, b3_ref,
                 g3_ref, t3_ref, o_ref):
    h = _dot_t(x_ref[...], w2_ref[...]) + b2_ref[...]
    h = jnp.maximum(h * (g2_ref[...] * INV) + t2_ref[...], 0.0)
    h2 = _dot_t(h, w3_ref[...]) + b3_ref[...]
    o_ref[...] = jnp.maximum(h2 * (g3_ref[...] * INV) + t3_ref[...], 0.0)


def _meta_body(wdp_ref, mg_ref, oh_ref, w0_ref, b0_ref, w1_ref, b1_ref,
               pw1_ref, pb1_ref, pg1_ref, pt1_ref, pw2_ref, pb2_ref,
               pg2_ref, pt2_ref, o_ref):
    wd = jnp.sum(wdp_ref[...], axis=0).reshape(NMET, NMET)
    deg = jnp.sum(wd, axis=1, keepdims=True)
    dis = jnp.where(deg > 0, lax.rsqrt(jnp.maximum(deg, 1e-12)), 0.0)
    dis2 = dis * dis

    def a2(v):
        t = dis * v
        t = lax.dot_general(wd, t, (((1,), (0,)), ((), ())),
                            preferred_element_type=jnp.float32)
        t = dis2 * t
        t = lax.dot_general(wd, t, (((1,), (0,)), ((), ())),
                            preferred_element_type=jnp.float32)
        return dis * t

    m = mg_ref[...]
    m = _dot_t(a2(m), w0_ref[...]) + b0_ref[...]
    m = jnp.maximum(m, 0.0)
    m = _dot_t(a2(m), w1_ref[...]) + b1_ref[...]
    pe = lax.dot_general(oh_ref[...], m, (((1,), (0,)), ((), ())),
                         preferred_element_type=jnp.float32)
    h = _dot_t(pe, pw1_ref[...]) + pb1_ref[...]
    h = jnp.maximum(h * (pg1_ref[...] * INV) + pt1_ref[...], 0.0)
    h2 = _dot_t(h, pw2_ref[...]) + pb2_ref[...]
    o_ref[...] = h2 * (pg2_ref[...] * INV) + pt2_ref[...]


def _pert_body(g_ref, pw1_ref, pb1_ref, pg1_ref, pt1_ref, pw2_ref, pb2_ref,
               pg2_ref, pt2_ref, o_ref):
    v = g_ref[...]
    rn = jnp.sqrt(jnp.sum(v * v, axis=1, keepdims=True))
    v = jnp.where(rn > 1.0, v / jnp.maximum(rn, 1e-12), v)
    r = lax.broadcasted_iota(jnp.int32, (B, 512), 0)
    k = lax.broadcasted_iota(jnp.int32, (B, 512), 1)
    sel = jnp.where(lax.div(k, 8) == r, 1.0, 0.0)
    sv = lax.dot_general(sel, v, (((1,), (0,)), ((), ())),
                         preferred_element_type=jnp.float32)
    h = _dot_t(sv, pw1_ref[...]) + pb1_ref[...]
    h = jnp.maximum(h * (pg1_ref[...] * INV) + pt1_ref[...], 0.0)
    h2 = _dot_t(h, pw2_ref[...]) + pb2_ref[...]
    o_ref[...] = h2 * (pg2_ref[...] * INV) + pt2_ref[...]


def _final_body(x_ref, pe_ref, pr_ref, w1_ref, b1_ref, w2_ref, b2_ref,
                w3_ref, b3_ref, w4_ref, b4_ref, wo_ref, bo_ref, o_ref):
    x = jnp.concatenate([x_ref[...], pe_ref[...], pr_ref[...]], axis=1)
    h = jnp.maximum(_dot_t(x, w1_ref[...]) + b1_ref[...], 0.0)
    h = jnp.maximum(_dot_t(h, w2_ref[...]) + b2_ref[...], 0.0)
    h = jnp.maximum(_dot_t(h, w3_ref[...]) + b3_ref[...], 0.0)
    h = _dot_t(h, w4_ref[...]) + b4_ref[...]
    z = _dot_t(h, wo_ref[...]) + bo_ref[...]
    m = jnp.max(z, axis=1, keepdims=True)
    e = jnp.exp(z - m)
    o_ref[...] = e / jnp.sum(e, axis=1, keepdims=True)


def kernel(edge_index, edge_weight, meta_edge_index, meta_edge_weight,
           product_idx, pert_index, batch, params):
    p = params
    f32 = jnp.float32

    ar = jnp.arange(N, dtype=jnp.int32)
    row = jnp.concatenate([edge_index[0], ar])
    col = jnp.concatenate([edge_index[1], ar])
    ew2 = jnp.concatenate([edge_weight, jnp.ones((N,), f32)])

    degp = _deg_kernel(col, ew2)
    dis64 = _tc_call(_dis_body, jax.ShapeDtypeStruct((N // 128, 128), f32))(
        degp.reshape(NW, N // 128, 128))
    dis = dis64.reshape(N)
    norm, cntf = _norm_kernel(row, col, ew2, dis)

    cnts = cntf.astype(jnp.int32)
    c16 = ((cnts + 15) // 16) * 16
    tot16 = jnp.sum(c16, axis=0)
    caps = ((tot16 + 255) // 256) * 256
    bases = jnp.concatenate([jnp.zeros((1,), jnp.int32),
                             jnp.cumsum(caps)[:-1].astype(jnp.int32)])
    excl = jnp.concatenate([jnp.zeros((1, NW), jnp.int32),
                            jnp.cumsum(c16, axis=0)[:-1].astype(jnp.int32)],
                           axis=0)
    padL = jnp.zeros((L,), jnp.int32)
    offs = jnp.concatenate([(bases[None, :] + excl).reshape(-1), padL])
    tzs = jnp.concatenate([bases + tot16, padL])
    tzn = jnp.concatenate([(caps - tot16) // 16, padL])
    nch = jnp.concatenate([caps // CH, padL])
    bases_p = jnp.concatenate([bases, padL])

    pk, nm = _bucket_kernel(row, col, norm, offs, tzs, tzn)

    mg_mx = _tc_call(_maxnorm_body, jax.ShapeDtypeStruct((NMET, H), f32))(
        p['meta_graph_emb'])
    def r1(v):
        return v.reshape(1, -1)

    h1 = _hop1_kernel(mg_mx, pk, nm, bases_p, nch)
    h2 = _hop_kernel(h1, pk, nm, bases_p, nch)
    grid16 = (16,)
    bs_x = pl.BlockSpec((512, H), lambda i: (i, 0))
    bs_w = pl.BlockSpec((H, H), lambda i: (0, 0))
    bs_b = pl.BlockSpec((1, H), lambda i: (0, 0))
    x1 = _tc_call(_linear_relu_body, jax.ShapeDtypeStruct((N, H), f32),
                  grid=grid16, in_specs=[bs_x, bs_w, bs_b],
                  out_specs=bs_x)(h2, p['sg_gem_0_W'], r1(p['sg_gem_0_b']))
    h3 = _hop_kernel(x1, pk, nm, bases_p, nch)
    h4 = _hop_kernel(h3, pk, nm, bases_p, nch)
    base_emb = _tc_call(
        _sg_embmlp_body, jax.ShapeDtypeStruct((N, H), f32), grid=grid16,
        in_specs=[bs_x, bs_w, bs_b] + [bs_w, bs_b, bs_b, bs_b] * 2,
        out_specs=bs_x)(
        h4, p['sg_gem_1_W'], r1(p['sg_gem_1_b']),
        p['emb_mlp_W1'], r1(p['emb_mlp_b1']), r1(p['emb_mlp_bn1_g']), r1(p['emb_mlp_bn1_b']),
        p['emb_mlp_W2'], r1(p['emb_mlp_b2']), r1(p['emb_mlp_bn2_g']), r1(p['emb_mlp_bn2_b']))

    xflat = base_emb.reshape(B, NMET * H)
    fl1 = _tc_call(
        _flat1_body, jax.ShapeDtypeStruct((B, 1024), f32), grid=(32,),
        in_specs=[pl.BlockSpec((B, 512), lambda k: (0, k)),
                  pl.BlockSpec((1024, 512), lambda k: (0, k)),
                  pl.BlockSpec((1, 1024), lambda k: (0, 0)),
                  pl.BlockSpec((1, 1024), lambda k: (0, 0)),
                  pl.BlockSpec((1, 1024), lambda k: (0, 0))],
        out_specs=pl.BlockSpec((B, 1024), lambda k: (0, 0)))(
        xflat, p['flat_fc1_W'], r1(p['flat_fc1_b']),
        r1(p['flat_bn1_g']), r1(p['flat_bn1_b']))
    fl3 = _tc_call(_flat23_body, jax.ShapeDtypeStruct((B, H), f32))(
        fl1, p['flat_fc2_W'], r1(p['flat_fc2_b']), r1(p['flat_bn2_g']), r1(p['flat_bn2_b']),
        p['flat_fc3_W'], r1(p['flat_fc3_b']), r1(p['flat_bn3_g']), r1(p['flat_bn3_b']))

    mar = jnp.arange(NMET, dtype=jnp.int32)
    pad = EM2P - (meta_edge_index.shape[1] + NMET)
    mrow = jnp.concatenate([meta_edge_index[0], mar, jnp.zeros((pad,), jnp.int32)])
    mcol = jnp.concatenate([meta_edge_index[1], mar, jnp.zeros((pad,), jnp.int32)])
    mew = jnp.concatenate([meta_edge_weight, jnp.ones((NMET,), f32),
                           jnp.zeros((pad,), f32)])
    wdp = _meta_adj_kernel(mrow, mcol, mew)
    onehot = (product_idx[:, None] == mar[None, :]).astype(f32)
    prod = _tc_call(_meta_body, jax.ShapeDtypeStruct((B, H), f32))(
        wdp.reshape(NW, NMET, NMET), mg_mx, onehot,
        p['sg_meta_0_W'], r1(p['sg_meta_0_b']),
        p['sg_meta_1_W'], r1(p['sg_meta_1_b']),
        p['product_mlp_W1'], r1(p['product_mlp_b1']),
        r1(p['product_mlp_bn1_g']), r1(p['product_mlp_bn1_b']),
        p['product_mlp_W2'], r1(p['product_mlp_b2']),
        r1(p['product_mlp_bn2_g']), r1(p['product_mlp_bn2_b']))

    pg = _pert_gather_kernel(p['pert_emb'], pert_index.reshape(512))
    pert = _tc_call(_pert_body, jax.ShapeDtypeStruct((B, H), f32))(
        pg,
        p['pert_mlp_W1'], r1(p['pert_mlp_b1']),
        r1(p['pert_mlp_bn1_g']), r1(p['pert_mlp_bn1_b']),
        p['pert_mlp_W2'], r1(p['pert_mlp_b2']),
        r1(p['pert_mlp_bn2_g']), r1(p['pert_mlp_bn2_b']))

    wo = jnp.zeros((128, 256), f32).at[:2].set(p['fc_out_W'])
    bo = jnp.full((1, 128), -1e30, f32).at[0, :2].set(p['fc_out_b'])
    out = _tc_call(_final_body, jax.ShapeDtypeStruct((B, 128), f32))(
        fl3, pert, prod,
        p['ff1_W'], r1(p['ff1_b']), p['ff2_W'], r1(p['ff2_b']),
        p['ff3_W'], r1(p['ff3_b']), p['ff4_W'], r1(p['ff4_b']),
        wo, bo)
    return out[:, :2]
